# baseline (speedup 1.0000x reference)
import functools

import jax
import jax.numpy as jnp
import numpy as np
from jax import lax
from jax.experimental import pallas as pl
from jax.experimental.pallas import tpu as pltpu
from jax.experimental.pallas import tpu_sc as plsc

N = 10000
B = 8
S = 2000
E = 160000
K = 512
NC = 10
G = 45 * 45

SPAD = 2048
EPAD = 176128
NBLK = EPAD // 256
DPAD = 16384
DBLK = DPAD // 256

_NWORK = 32
_EPW = EPAD // _NWORK
_CH = 128
_NCH = _EPW // _CH
_SPW = SPAD // _NWORK


def _sc_gather(tbl16, srcp, cidx, idxp):
    mesh = plsc.VectorSubcoreMesh(core_axis_name="c", subcore_axis_name="s")

    @functools.partial(
        pl.kernel,
        out_type=(
            jax.ShapeDtypeStruct((EPAD, 16), jnp.float32),
            jax.ShapeDtypeStruct((EPAD, 16), jnp.float32),
            jax.ShapeDtypeStruct((SPAD, 16), jnp.float32),
        ),
        mesh=mesh,
        scratch_types=[
            pltpu.VMEM((_EPW,), jnp.int32),
            pltpu.VMEM((_EPW, 16), jnp.float32),
            pltpu.VMEM((_SPW, 16), jnp.float32),
            pltpu.SemaphoreType.DMA,
        ],
        compiler_params=pltpu.CompilerParams(use_tc_tiling_on_sc=False),
    )
    def k(tbl_hbm, srcp_hbm, cidx_hbm, idxp_hbm, a_hbm, b_hbm, poss_hbm,
          si_v, big_v, ps_v, sem):
        wid = lax.axis_index("s") * 2 + lax.axis_index("c")
        base = wid * _EPW

        def gather_big(idx_hbm, dst_hbm):
            pltpu.sync_copy(idx_hbm.at[pl.ds(base, _EPW)], si_v)

            def fire(ci, carry):
                off = ci * _CH
                pltpu.async_copy(
                    tbl_hbm.at[si_v.at[pl.ds(off, _CH)]],
                    big_v.at[pl.ds(off, _CH), :], sem)
                return carry

            lax.fori_loop(0, _NCH, fire, 0)
            pltpu.make_async_copy(
                tbl_hbm.at[pl.ds(0, _EPW), :], big_v, sem).wait()
            pltpu.sync_copy(big_v, dst_hbm.at[pl.ds(base, _EPW), :])

        gather_big(srcp_hbm, a_hbm)
        gather_big(cidx_hbm, b_hbm)

        sbase = wid * _SPW
        pltpu.sync_copy(idxp_hbm.at[pl.ds(sbase, _SPW)], si_v.at[pl.ds(0, _SPW)])
        pltpu.async_copy(
            tbl_hbm.at[si_v.at[pl.ds(0, _SPW)]], ps_v, sem).wait()
        pltpu.sync_copy(ps_v, poss_hbm.at[pl.ds(sbase, _SPW), :])

    return k(tbl16, srcp, cidx, idxp)


def _mlp_segmax(a, b, gkeys, w1, b1, w2, b2, w3, b3):
    def body(keys_ref, a_ref, b_ref, w1_ref, b1_ref, w2_ref, b2_ref,
             w3_ref, b3_ref, out_ref, gm_ref):
        i = pl.program_id(0)

        @pl.when(i == 0)
        def _init():
            out_ref[...] = jnp.full((SPAD, 512), -jnp.inf, jnp.float32)

        msg = a_ref[...] - b_ref[...]
        h = jnp.dot(msg, w1_ref[...], preferred_element_type=jnp.float32)
        h = h + b1_ref[...]
        h = jnp.where(h > 0, h, 0.2 * h)
        h = jnp.dot(h, w2_ref[...], preferred_element_type=jnp.float32)
        h = h + b2_ref[...]
        h = jnp.where(h > 0, h, 0.2 * h)
        z = jnp.dot(h, w3_ref[...], preferred_element_type=jnp.float32)
        z = z + b3_ref[...]
        z = jnp.where(z > 0, z, 0.2 * z)
        gm_ref[...] = jnp.max(z.reshape(32, 8, 512), axis=1)

        def upd(j, carry):
            d = keys_ref[0, 0, j]
            row = gm_ref[pl.ds(j, 1), :]
            out_ref[pl.ds(d, 1), :] = jnp.maximum(out_ref[pl.ds(d, 1), :], row)
            return carry

        lax.fori_loop(0, 32, upd, 0)

    return pl.pallas_call(
        body,
        grid=(NBLK,),
        in_specs=[
            pl.BlockSpec((1, 1, 32), lambda i: (i, 0, 0),
                         memory_space=pltpu.SMEM),
            pl.BlockSpec((256, 16), lambda i: (i, 0)),
            pl.BlockSpec((256, 16), lambda i: (i, 0)),
            pl.BlockSpec((16, 64), lambda i: (0, 0)),
            pl.BlockSpec((1, 64), lambda i: (0, 0)),
            pl.BlockSpec((64, 128), lambda i: (0, 0)),
            pl.BlockSpec((1, 128), lambda i: (0, 0)),
            pl.BlockSpec((128, 512), lambda i: (0, 0)),
            pl.BlockSpec((1, 512), lambda i: (0, 0)),
        ],
        out_specs=pl.BlockSpec((SPAD, 512), lambda i: (0, 0)),
        out_shape=jax.ShapeDtypeStruct((SPAD, 512), jnp.float32),
        scratch_shapes=[pltpu.VMEM((32, 512), jnp.float32)],
        compiler_params=pltpu.CompilerParams(
            dimension_semantics=("arbitrary",)),
    )(gkeys, a, b, w1, b1, w2, b2, w3, b3)


def _encode_select(aggraw, poss, ew1a, ew1b, eb1, ew2, eb2):
    def body(agg_ref, poss_ref, w1a_ref, w1b_ref, b1_ref, w2_ref, b2_ref,
             z_ref, mi_ref, is_ref, sb_ref):
        raw = agg_ref[...]
        agg = jnp.where(raw < -1e30, 0.0, raw)
        p16 = poss_ref[...]
        h2 = (jnp.dot(agg, w1a_ref[...], preferred_element_type=jnp.float32)
              + jnp.dot(p16, w1b_ref[...], preferred_element_type=jnp.float32)
              + b1_ref[...])
        h2 = jnp.where(h2 > 0, h2, 0.2 * h2)
        out = jnp.dot(h2, w2_ref[...], preferred_element_type=jnp.float32)
        out = out + b2_ref[...]
        mean = out[:, :512]
        lv = out[:, 512:]
        invstd = jnp.exp(-0.5 * lv)
        std = jnp.exp(0.5 * lv)
        mi_ref[...] = mean * invstd
        is_ref[...] = invstd
        score = jnp.mean(std, axis=1, keepdims=True)
        rowid = lax.broadcasted_iota(jnp.int32, (SPAD, 1), 0)
        bcol = p16[:, 3:4]
        valid = rowid < S
        for bb in range(B):
            act0 = jnp.logical_and(valid, bcol == float(bb))
            sb_ref[...] = jnp.where(act0, score, jnp.inf)

            def rnd(r, carry):
                nu, de = carry
                cur = sb_ref[...]
                m = jnp.min(cur)
                cand = jnp.where(cur == m, rowid, jnp.int32(2 ** 30))
                isel = jnp.min(cand)
                nu = nu + mi_ref[pl.ds(isel, 1), :]
                de = de + is_ref[pl.ds(isel, 1), :]
                sb_ref[pl.ds(isel, 1), :] = jnp.full((1, 1), jnp.inf,
                                                     jnp.float32)
                return nu, de

            nu, de = lax.fori_loop(
                0, NC, rnd,
                (jnp.zeros((1, 512), jnp.float32),
                 jnp.zeros((1, 512), jnp.float32)))
            z_ref[bb:bb + 1, :] = nu / de

    return pl.pallas_call(
        body,
        grid=(1,),
        in_specs=[
            pl.BlockSpec((SPAD, 512), lambda i: (0, 0)),
            pl.BlockSpec((SPAD, 16), lambda i: (0, 0)),
            pl.BlockSpec((512, 512), lambda i: (0, 0)),
            pl.BlockSpec((16, 512), lambda i: (0, 0)),
            pl.BlockSpec((1, 512), lambda i: (0, 0)),
            pl.BlockSpec((512, 1024), lambda i: (0, 0)),
            pl.BlockSpec((1, 1024), lambda i: (0, 0)),
        ],
        out_specs=pl.BlockSpec((8, 512), lambda i: (0, 0)),
        out_shape=jax.ShapeDtypeStruct((8, 512), jnp.float32),
        scratch_shapes=[
            pltpu.VMEM((SPAD, 512), jnp.float32),
            pltpu.VMEM((SPAD, 512), jnp.float32),
            pltpu.VMEM((SPAD, 1), jnp.float32),
        ],
        compiler_params=pltpu.CompilerParams(
            vmem_limit_bytes=100 * 1024 * 1024),
    )(aggraw, poss, ew1a, ew1b, eb1, ew2, eb2)


def _decode(z, gg, f1w1a, f1w1g, f1b1, f1w2, f1b2, f1w3p, f1b3p,
            f2w1a, f2w1b, f2b1, f2w2, f2b2, f2w3p, f2b3p):
    def body(z_ref, gg_ref, w1a, w1g, bb1, w12, b12, w13, b13,
             w2a, w2b, bb2, w22, b22, w23, b23, out_ref):
        zf = z_ref[...]
        gg_ = gg_ref[...]
        bcol = gg_[:, 3:4]
        code = jnp.zeros((256, 512), jnp.float32)
        for bb in range(B):
            code = jnp.where(bcol == float(bb), zf[bb:bb + 1, :], code)
        h = (jnp.dot(code, w1a[...], preferred_element_type=jnp.float32)
             + jnp.dot(gg_, w1g[...], preferred_element_type=jnp.float32)
             + bb1[...])
        h = jnp.maximum(h, 0.0)
        h = jnp.dot(h, w12[...], preferred_element_type=jnp.float32) + b12[...]
        h = jnp.maximum(h, 0.0)
        x1 = jnp.dot(h, w13[...], preferred_element_type=jnp.float32) + b13[...]
        h = (jnp.dot(code, w2a[...], preferred_element_type=jnp.float32)
             + jnp.dot(x1, w2b[...], preferred_element_type=jnp.float32)
             + bb2[...])
        h = jnp.maximum(h, 0.0)
        h = jnp.dot(h, w22[...], preferred_element_type=jnp.float32) + b22[...]
        h = jnp.maximum(h, 0.0)
        out_ref[...] = (jnp.dot(h, w23[...], preferred_element_type=jnp.float32)
                        + b23[...])

    wspec = lambda r, c: pl.BlockSpec((r, c), lambda i: (0, 0))
    return pl.pallas_call(
        body,
        grid=(DBLK,),
        in_specs=[
            pl.BlockSpec((8, 512), lambda i: (0, 0)),
            pl.BlockSpec((256, 16), lambda i: (i, 0)),
            wspec(512, 512), wspec(16, 512), wspec(1, 512),
            wspec(512, 512), wspec(1, 512),
            wspec(512, 16), wspec(1, 16),
            wspec(512, 512), wspec(16, 512), wspec(1, 512),
            wspec(512, 512), wspec(1, 512),
            wspec(512, 16), wspec(1, 16),
        ],
        out_specs=pl.BlockSpec((256, 16), lambda i: (i, 0)),
        out_shape=jax.ShapeDtypeStruct((DPAD, 16), jnp.float32),
        compiler_params=pltpu.CompilerParams(
            dimension_semantics=("arbitrary",)),
    )(z, gg, f1w1a, f1w1g, f1b1, f1w2, f1b2, f1w3p, f1b3p,
      f2w1a, f2w1b, f2b1, f2w2, f2b2, f2w3p, f2b3p)


def _np_grid16():
    g1, g2 = np.meshgrid(np.linspace(-0.3, 0.3, 45), np.linspace(-0.3, 0.3, 45))
    g = np.stack([g1.reshape(-1), g2.reshape(-1)], axis=-1).astype(np.float32)
    gg = np.zeros((DPAD, 16), np.float32)
    t = np.arange(DPAD)
    bcol = np.minimum(t // G, B - 1)
    gg[:B * G, 0:2] = np.tile(g, (B, 1))
    gg[:, 3] = bcol
    return gg


_GG16_NP = _np_grid16()


def kernel(pos, batch, idx, src, dst, lW1, lb1, lW2, lb2, lW3, lb3,
           eW1, eb1, eW2, eb2,
           f1W1, f1b1, f1W2, f1b2, f1W3, f1b3,
           f2W1, f2b1, f2W2, f2b2, f2W3, f2b3):
    i32 = jnp.int32
    order = jnp.argsort(dst)
    dst_s = dst[order].astype(i32)
    src_s = src[order].astype(i32)
    bounds = jnp.searchsorted(dst_s, jnp.arange(S + 1, dtype=i32)).astype(i32)
    counts = bounds[1:] - bounds[:-1]
    pcounts = (counts + 7) // 8 * 8
    poff = jnp.concatenate([jnp.zeros((1,), i32), jnp.cumsum(pcounts)])
    total = poff[S]
    t = jnp.arange(EPAD, dtype=i32)
    seg = jnp.clip(jnp.searchsorted(poff, t, side='right').astype(i32) - 1,
                   0, S - 1)
    local = t - poff[seg]
    ecl = jnp.minimum(local, counts[seg] - 1)
    valid = t < total
    esel = jnp.where(valid, bounds[seg] + ecl, 0)
    key = dst_s[esel]
    srcp = src_s[esel]
    cidx = idx[key].astype(i32)
    gkeys = key[::8].reshape(NBLK, 1, 32)
    idxp = jnp.concatenate([idx.astype(i32), jnp.zeros((SPAD - S,), i32)])

    tbl16 = jnp.concatenate(
        [pos, batch.astype(jnp.float32)[:, None],
         jnp.zeros((N, 12), jnp.float32)], axis=1)

    a, b, poss = _sc_gather(tbl16, srcp, cidx, idxp)

    w1 = jnp.zeros((16, 64), jnp.float32).at[:3].set(lW1)
    aggraw = _mlp_segmax(a, b, gkeys, w1, lb1[None, :], lW2, lb2[None, :],
                         lW3, lb3[None, :])

    ew1a = eW1[:512]
    ew1b = jnp.zeros((16, 512), jnp.float32).at[:3].set(eW1[512:515])
    z = _encode_select(aggraw, poss, ew1a, ew1b, eb1[None, :], eW2,
                       eb2[None, :])

    f1w1a = f1W1[:512]
    f1w1g = jnp.zeros((16, 512), jnp.float32).at[:2].set(f1W1[512:514])
    f1w3p = jnp.zeros((512, 16), jnp.float32).at[:, :3].set(f1W3)
    f1b3p = jnp.zeros((1, 16), jnp.float32).at[0, :3].set(f1b3)
    f2w1a = f2W1[:512]
    f2w1b = jnp.zeros((16, 512), jnp.float32).at[:3].set(f2W1[512:515])
    f2w3p = jnp.zeros((512, 16), jnp.float32).at[:, :3].set(f2W3)
    f2b3p = jnp.zeros((1, 16), jnp.float32).at[0, :3].set(f2b3)
    outp = _decode(z, jnp.asarray(_GG16_NP), f1w1a, f1w1g, f1b1[None, :], f1W2, f1b2[None, :],
                   f1w3p, f1b3p, f2w1a, f2w1b, f2b1[None, :], f2W2,
                   f2b2[None, :], f2w3p, f2b3p)
    return outp[:B * G, :3].reshape(B, G, 3)


# speedup vs baseline: 16.6457x; 16.6457x over previous
import functools

import jax
import jax.numpy as jnp
import numpy as np
from jax import lax
from jax.experimental import pallas as pl
from jax.experimental.pallas import tpu as pltpu
from jax.experimental.pallas import tpu_sc as plsc

N = 10000
B = 8
S = 2000
E = 160000
K = 512
NC = 10
G = 45 * 45

SPAD = 2048
EP = 163840
EPAD = 178176
NBLK = EPAD // 256
NG = EPAD // 8
DPAD = 16384
DBLK = DPAD // 256

_NWORK = 32
_EPW = EP // _NWORK
_NV = _EPW // 16
_NCH = _EPW // 128
_SPW = SPAD // _NWORK

_MESH = dict(core_axis_name="c", subcore_axis_name="s")


def _sc_hist(dstp, tbl16, idxp):
    mesh = plsc.VectorSubcoreMesh(**_MESH)

    @functools.partial(
        pl.kernel,
        out_type=(
            jax.ShapeDtypeStruct((_NWORK, SPAD), jnp.int32),
            jax.ShapeDtypeStruct((SPAD, 16), jnp.float32),
        ),
        mesh=mesh,
        scratch_types=[
            pltpu.VMEM((_EPW,), jnp.int32),
            pltpu.VMEM((16 * SPAD,), jnp.int32),
            pltpu.VMEM((SPAD,), jnp.int32),
            pltpu.VMEM((_SPW, 16), jnp.float32),
            pltpu.VMEM((_SPW,), jnp.int32),
            pltpu.SemaphoreType.DMA,
        ],
        compiler_params=pltpu.CompilerParams(use_tc_tiling_on_sc=False, needs_layout_passes=False),
    )
    def k(dst_hbm, tbl_hbm, idxp_hbm, hist_hbm, poss_hbm,
          d_v, bank_v, tot_v, ps_v, pi_v, sem):
        wid = lax.axis_index("s") * 2 + lax.axis_index("c")
        base = wid * _EPW
        pltpu.sync_copy(dst_hbm.at[pl.ds(base, _EPW)], d_v)
        zero16 = jnp.zeros((16,), jnp.int32)

        def mz(i, c):
            bank_v[pl.ds(i * 16, 16)] = zero16
            return c

        lax.fori_loop(0, SPAD, mz, 0)
        lanes = lax.iota(jnp.int32, 16) * SPAD

        def acc(i, c):
            d = d_v[pl.ds(i * 16, 16)]
            addr = lanes + d
            old = plsc.load_gather(bank_v, [addr])
            plsc.store_scatter(bank_v, [addr], old + 1)
            return c

        lax.fori_loop(0, _NV, acc, 0)

        def red(j, c):
            sacc = jnp.zeros((16,), jnp.int32)
            for l in range(16):
                sacc = sacc + bank_v[pl.ds(l * SPAD + j * 16, 16)]
            tot_v[pl.ds(j * 16, 16)] = sacc
            return c

        lax.fori_loop(0, SPAD // 16, red, 0)
        pltpu.sync_copy(tot_v, hist_hbm.at[wid])

        sbase = wid * _SPW
        pltpu.sync_copy(idxp_hbm.at[pl.ds(sbase, _SPW)], pi_v)
        pltpu.async_copy(tbl_hbm.at[pi_v], ps_v, sem).wait()
        pltpu.sync_copy(ps_v, poss_hbm.at[pl.ds(sbase, _SPW), :])

    return k(dstp, tbl16, idxp)


def _sc_permute(dstp, srcp, tbase, tbl16, idxp):
    mesh = plsc.VectorSubcoreMesh(**_MESH)

    @functools.partial(
        pl.kernel,
        out_type=(
            jax.ShapeDtypeStruct((EPAD, 16), jnp.float32),
            jax.ShapeDtypeStruct((EPAD, 16), jnp.float32),
        ),
        mesh=mesh,
        scratch_types=[
            pltpu.VMEM((_EPW,), jnp.int32),
            pltpu.VMEM((_EPW,), jnp.int32),
            pltpu.VMEM((_EPW,), jnp.int32),
            pltpu.VMEM((_NCH, 128), jnp.int32),
            pltpu.VMEM((SPAD,), jnp.int32),
            pltpu.VMEM((SPAD,), jnp.int32),
            pltpu.VMEM((48,), jnp.int32),
            pltpu.VMEM((_EPW, 16), jnp.float32),
            pltpu.SemaphoreType.DMA,
        ],
        compiler_params=pltpu.CompilerParams(use_tc_tiling_on_sc=False, needs_layout_passes=False),
    )
    def k(dst_hbm, src_hbm, tb_hbm, tbl_hbm, idxp_hbm, a_hbm, b_hbm,
          d_v, s_v, c_v, slot_v, next_v, idx_v, sh_v, rows_v, sem):
        wid = lax.axis_index("s") * 2 + lax.axis_index("c")
        base = wid * _EPW
        pltpu.sync_copy(dst_hbm.at[pl.ds(base, _EPW)], d_v)
        pltpu.sync_copy(src_hbm.at[pl.ds(base, _EPW)], s_v)
        pltpu.sync_copy(tb_hbm.at[wid], next_v)
        pltpu.sync_copy(idxp_hbm, idx_v)
        neg16 = jnp.full((16,), -1, jnp.int32)
        sh_v[pl.ds(0, 16)] = neg16
        sh_v[pl.ds(32, 16)] = neg16
        pos = lax.iota(jnp.int32, 16)

        def vstep(i, c):
            d = d_v[pl.ds(i * 16, 16)]
            c_v[pl.ds(i * 16, 16)] = plsc.load_gather(idx_v, [d])
            sd, sl = plsc.sort_key_val(d, pos)
            sh_v[pl.ds(16, 16)] = sd
            prev = sh_v[pl.ds(15, 16)]
            nxt = sh_v[pl.ds(17, 16)]
            head = sd != prev
            tail = sd != nxt
            hp = plsc.cummax(jnp.where(head, pos, 0))
            r = pos - hp
            old = plsc.load_gather(next_v, [sd])
            plsc.store_scatter(next_v, [sd], old + r + 1, mask=tail)
            ch = i // 8
            kk = i % 8
            plsc.store_scatter(
                slot_v, [jnp.full((16,), ch, jnp.int32), kk * 16 + sl],
                old + r)
            return c

        lax.fori_loop(0, _NV, vstep, 0)

        def pump(ids_v, out_hbm):
            def fireg(ci, c):
                off = ci * 128
                pltpu.async_copy(tbl_hbm.at[ids_v.at[pl.ds(off, 128)]],
                                 rows_v.at[pl.ds(off, 128), :], sem)
                return c

            lax.fori_loop(0, _NCH, fireg, 0)
            pltpu.make_async_copy(
                tbl_hbm.at[pl.ds(0, _EPW), :], rows_v, sem).wait()

            def fires(ci, c):
                off = ci * 128
                pltpu.async_copy(rows_v.at[pl.ds(off, 128), :],
                                 out_hbm.at[slot_v.at[ci]], sem)
                return c

            lax.fori_loop(0, _NCH, fires, 0)
            pltpu.make_async_copy(
                out_hbm.at[pl.ds(0, _EPW), :], rows_v, sem).wait()

        pump(s_v, a_hbm)
        pump(c_v, b_hbm)

    return k(dstp, srcp, tbase, tbl16, idxp)


def _mlp_segmax(a, b, vrow, gkeys, w1, b1, w2, b2, w3, b3):
    def body(keys_ref, a_ref, b_ref, v_ref, w1_ref, b1_ref, w2_ref, b2_ref,
             w3_ref, b3_ref, out_ref, gm_ref):
        i = pl.program_id(0)

        @pl.when(i == 0)
        def _init():
            out_ref[...] = jnp.full((SPAD, 512), -jnp.inf, jnp.float32)

        msg = a_ref[...] - b_ref[...]
        h = jnp.dot(msg, w1_ref[...], preferred_element_type=jnp.float32)
        h = h + b1_ref[...]
        h = jnp.where(h > 0, h, 0.2 * h)
        h = jnp.dot(h, w2_ref[...], preferred_element_type=jnp.float32)
        h = h + b2_ref[...]
        h = jnp.where(h > 0, h, 0.2 * h)
        z = jnp.dot(h, w3_ref[...], preferred_element_type=jnp.float32)
        z = z + b3_ref[...]
        z = jnp.where(z > 0, z, 0.2 * z)
        z = jnp.where(v_ref[...] > 0, z, -jnp.inf)
        gm_ref[...] = jnp.max(z.reshape(32, 8, 512), axis=1)

        def upd(j, carry):
            d = keys_ref[0, 0, j]
            row = gm_ref[pl.ds(j, 1), :]
            out_ref[pl.ds(d, 1), :] = jnp.maximum(out_ref[pl.ds(d, 1), :], row)
            return carry

        lax.fori_loop(0, 32, upd, 0)

    return pl.pallas_call(
        body,
        grid=(NBLK,),
        in_specs=[
            pl.BlockSpec((1, 1, 32), lambda i: (i, 0, 0),
                         memory_space=pltpu.SMEM),
            pl.BlockSpec((256, 16), lambda i: (i, 0)),
            pl.BlockSpec((256, 16), lambda i: (i, 0)),
            pl.BlockSpec((256, 1), lambda i: (i, 0)),
            pl.BlockSpec((16, 64), lambda i: (0, 0)),
            pl.BlockSpec((1, 64), lambda i: (0, 0)),
            pl.BlockSpec((64, 128), lambda i: (0, 0)),
            pl.BlockSpec((1, 128), lambda i: (0, 0)),
            pl.BlockSpec((128, 512), lambda i: (0, 0)),
            pl.BlockSpec((1, 512), lambda i: (0, 0)),
        ],
        out_specs=pl.BlockSpec((SPAD, 512), lambda i: (0, 0)),
        out_shape=jax.ShapeDtypeStruct((SPAD, 512), jnp.float32),
        scratch_shapes=[pltpu.VMEM((32, 512), jnp.float32)],
        compiler_params=pltpu.CompilerParams(
            dimension_semantics=("arbitrary",)),
    )(gkeys, a, b, vrow, w1, b1, w2, b2, w3, b3)


def _encode_select(aggraw, poss, ew1a, ew1b, eb1, ew2, eb2):
    def body(agg_ref, poss_ref, w1a_ref, w1b_ref, b1_ref, w2_ref, b2_ref,
             z_ref, mi_ref, is_ref, sb_ref):
        raw = agg_ref[...]
        agg = jnp.where(raw < -1e30, 0.0, raw)
        p16 = poss_ref[...]
        h2 = (jnp.dot(agg, w1a_ref[...], preferred_element_type=jnp.float32)
              + jnp.dot(p16, w1b_ref[...], preferred_element_type=jnp.float32)
              + b1_ref[...])
        h2 = jnp.where(h2 > 0, h2, 0.2 * h2)
        out = jnp.dot(h2, w2_ref[...], preferred_element_type=jnp.float32)
        out = out + b2_ref[...]
        mean = out[:, :512]
        lv = out[:, 512:]
        invstd = jnp.exp(-0.5 * lv)
        std = jnp.exp(0.5 * lv)
        mi_ref[...] = mean * invstd
        is_ref[...] = invstd
        score = jnp.mean(std, axis=1, keepdims=True)
        rowid = lax.broadcasted_iota(jnp.int32, (SPAD, 1), 0)
        bcol = p16[:, 3:4]
        valid = rowid < S
        for bb in range(B):
            act0 = jnp.logical_and(valid, bcol == float(bb))
            sb_ref[...] = jnp.where(act0, score, jnp.inf)

            def rnd(r, carry):
                nu, de = carry
                cur = sb_ref[...]
                m = jnp.min(cur)
                cand = jnp.where(cur == m, rowid, jnp.int32(2 ** 30))
                isel = jnp.min(cand)
                nu = nu + mi_ref[pl.ds(isel, 1), :]
                de = de + is_ref[pl.ds(isel, 1), :]
                sb_ref[pl.ds(isel, 1), :] = jnp.full((1, 1), jnp.inf,
                                                     jnp.float32)
                return nu, de

            nu, de = lax.fori_loop(
                0, NC, rnd,
                (jnp.zeros((1, 512), jnp.float32),
                 jnp.zeros((1, 512), jnp.float32)))
            z_ref[bb:bb + 1, :] = nu / de

    return pl.pallas_call(
        body,
        grid=(1,),
        in_specs=[
            pl.BlockSpec((SPAD, 512), lambda i: (0, 0)),
            pl.BlockSpec((SPAD, 16), lambda i: (0, 0)),
            pl.BlockSpec((512, 512), lambda i: (0, 0)),
            pl.BlockSpec((16, 512), lambda i: (0, 0)),
            pl.BlockSpec((1, 512), lambda i: (0, 0)),
            pl.BlockSpec((512, 1024), lambda i: (0, 0)),
            pl.BlockSpec((1, 1024), lambda i: (0, 0)),
        ],
        out_specs=pl.BlockSpec((8, 512), lambda i: (0, 0)),
        out_shape=jax.ShapeDtypeStruct((8, 512), jnp.float32),
        scratch_shapes=[
            pltpu.VMEM((SPAD, 512), jnp.float32),
            pltpu.VMEM((SPAD, 512), jnp.float32),
            pltpu.VMEM((SPAD, 1), jnp.float32),
        ],
        compiler_params=pltpu.CompilerParams(
            vmem_limit_bytes=100 * 1024 * 1024),
    )(aggraw, poss, ew1a, ew1b, eb1, ew2, eb2)


def _decode(z, gg, f1w1a, f1w1g, f1b1, f1w2, f1b2, f1w3p, f1b3p,
            f2w1a, f2w1b, f2b1, f2w2, f2b2, f2w3p, f2b3p):
    def body(z_ref, gg_ref, w1a, w1g, bb1, w12, b12, w13, b13,
             w2a, w2b, bb2, w22, b22, w23, b23, out_ref):
        zf = z_ref[...]
        gg_ = gg_ref[...]
        bcol = gg_[:, 3:4]
        code = jnp.zeros((256, 512), jnp.float32)
        for bb in range(B):
            code = jnp.where(bcol == float(bb), zf[bb:bb + 1, :], code)
        h = (jnp.dot(code, w1a[...], preferred_element_type=jnp.float32)
             + jnp.dot(gg_, w1g[...], preferred_element_type=jnp.float32)
             + bb1[...])
        h = jnp.maximum(h, 0.0)
        h = jnp.dot(h, w12[...], preferred_element_type=jnp.float32) + b12[...]
        h = jnp.maximum(h, 0.0)
        x1 = jnp.dot(h, w13[...], preferred_element_type=jnp.float32) + b13[...]
        h = (jnp.dot(code, w2a[...], preferred_element_type=jnp.float32)
             + jnp.dot(x1, w2b[...], preferred_element_type=jnp.float32)
             + bb2[...])
        h = jnp.maximum(h, 0.0)
        h = jnp.dot(h, w22[...], preferred_element_type=jnp.float32) + b22[...]
        h = jnp.maximum(h, 0.0)
        out_ref[...] = (jnp.dot(h, w23[...], preferred_element_type=jnp.float32)
                        + b23[...])

    wspec = lambda r, c: pl.BlockSpec((r, c), lambda i: (0, 0))
    return pl.pallas_call(
        body,
        grid=(DBLK,),
        in_specs=[
            pl.BlockSpec((8, 512), lambda i: (0, 0)),
            pl.BlockSpec((256, 16), lambda i: (i, 0)),
            wspec(512, 512), wspec(16, 512), wspec(1, 512),
            wspec(512, 512), wspec(1, 512),
            wspec(512, 16), wspec(1, 16),
            wspec(512, 512), wspec(16, 512), wspec(1, 512),
            wspec(512, 512), wspec(1, 512),
            wspec(512, 16), wspec(1, 16),
        ],
        out_specs=pl.BlockSpec((256, 16), lambda i: (i, 0)),
        out_shape=jax.ShapeDtypeStruct((DPAD, 16), jnp.float32),
        compiler_params=pltpu.CompilerParams(
            dimension_semantics=("arbitrary",)),
    )(z, gg, f1w1a, f1w1g, f1b1, f1w2, f1b2, f1w3p, f1b3p,
      f2w1a, f2w1b, f2b1, f2w2, f2b2, f2w3p, f2b3p)


def _np_grid16():
    g1, g2 = np.meshgrid(np.linspace(-0.3, 0.3, 45), np.linspace(-0.3, 0.3, 45))
    g = np.stack([g1.reshape(-1), g2.reshape(-1)], axis=-1).astype(np.float32)
    gg = np.zeros((DPAD, 16), np.float32)
    t = np.arange(DPAD)
    bcol = np.minimum(t // G, B - 1)
    gg[:B * G, 0:2] = np.tile(g, (B, 1))
    gg[:, 3] = bcol
    return gg


_GG16_NP = _np_grid16()


def kernel(pos, batch, idx, src, dst, lW1, lb1, lW2, lb2, lW3, lb3,
           eW1, eb1, eW2, eb2,
           f1W1, f1b1, f1W2, f1b2, f1W3, f1b3,
           f2W1, f2b1, f2W2, f2b2, f2W3, f2b3):
    i32 = jnp.int32
    dstp = jnp.concatenate([dst.astype(i32),
                            jnp.full((EP - E,), SPAD - 1, i32)])
    srcp = jnp.concatenate([src.astype(i32), jnp.zeros((EP - E,), i32)])
    idxp = jnp.concatenate([idx.astype(i32), jnp.zeros((SPAD - S,), i32)])
    tbl16 = jnp.concatenate(
        [pos, batch.astype(jnp.float32)[:, None],
         jnp.zeros((N, 12), jnp.float32)], axis=1)

    hists, poss = _sc_hist(dstp, tbl16, idxp)

    counts = jnp.sum(hists, axis=0)
    pc8 = (counts + 7) // 8 * 8
    csum = jnp.cumsum(pc8)
    poff8 = jnp.concatenate([jnp.zeros((1,), i32), csum[:-1]])
    tbase = poff8[None, :] + (jnp.cumsum(hists, axis=0) - hists)
    gseg = jnp.repeat(jnp.arange(SPAD, dtype=i32), pc8 // 8,
                      total_repeat_length=NG)
    vcnt = jnp.clip(poff8[gseg] + counts[gseg]
                    - 8 * jnp.arange(NG, dtype=i32), 0, 8)
    vrow = (jnp.tile(jnp.arange(8, dtype=i32), NG)
            < jnp.repeat(vcnt, 8)).astype(jnp.float32)[:, None]
    gkeys = gseg.reshape(NBLK, 1, 32)

    a, b = _sc_permute(dstp, srcp, tbase, tbl16, idxp)

    w1 = jnp.zeros((16, 64), jnp.float32).at[:3].set(lW1)
    aggraw = _mlp_segmax(a, b, vrow, gkeys, w1, lb1[None, :], lW2,
                         lb2[None, :], lW3, lb3[None, :])

    ew1a = eW1[:512]
    ew1b = jnp.zeros((16, 512), jnp.float32).at[:3].set(eW1[512:515])
    z = _encode_select(aggraw, poss, ew1a, ew1b, eb1[None, :], eW2,
                       eb2[None, :])

    f1w1a = f1W1[:512]
    f1w1g = jnp.zeros((16, 512), jnp.float32).at[:2].set(f1W1[512:514])
    f1w3p = jnp.zeros((512, 16), jnp.float32).at[:, :3].set(f1W3)
    f1b3p = jnp.zeros((1, 16), jnp.float32).at[0, :3].set(f1b3)
    f2w1a = f2W1[:512]
    f2w1b = jnp.zeros((16, 512), jnp.float32).at[:3].set(f2W1[512:515])
    f2w3p = jnp.zeros((512, 16), jnp.float32).at[:, :3].set(f2W3)
    f2b3p = jnp.zeros((1, 16), jnp.float32).at[0, :3].set(f2b3)
    outp = _decode(z, jnp.asarray(_GG16_NP), f1w1a, f1w1g, f1b1[None, :],
                   f1W2, f1b2[None, :], f1w3p, f1b3p, f2w1a, f2w1b,
                   f2b1[None, :], f2W2, f2b2[None, :], f2w3p, f2b3p)
    return outp[:B * G, :3].reshape(B, G, 3)


# speedup vs baseline: 20.2073x; 1.2140x over previous
import functools

import jax
import jax.numpy as jnp
import numpy as np
from jax import lax
from jax.experimental import pallas as pl
from jax.experimental.pallas import tpu as pltpu
from jax.experimental.pallas import tpu_sc as plsc

N = 10000
B = 8
S = 2000
E = 160000
K = 512
NC = 10
G = 45 * 45

SPAD = 2048
EP = 163840
EPAD = 178176
NBLK = EPAD // 256
NG = EPAD // 8
DPAD = 16384
DBLK = DPAD // 256

_NWORK = 32
_EPW = EP // _NWORK
_NV = _EPW // 16
_NCH = _EPW // 128
_SPW = SPAD // _NWORK

_MESH = dict(core_axis_name="c", subcore_axis_name="s")


def _sc_hist(dstp, tbl16, idxp):
    mesh = plsc.VectorSubcoreMesh(**_MESH)

    @functools.partial(
        pl.kernel,
        out_type=(
            jax.ShapeDtypeStruct((_NWORK, SPAD), jnp.int32),
            jax.ShapeDtypeStruct((SPAD, 16), jnp.float32),
        ),
        mesh=mesh,
        scratch_types=[
            pltpu.VMEM((_EPW,), jnp.int32),
            pltpu.VMEM((16 * SPAD,), jnp.int32),
            pltpu.VMEM((SPAD,), jnp.int32),
            pltpu.VMEM((_SPW, 16), jnp.float32),
            pltpu.VMEM((_SPW,), jnp.int32),
            pltpu.SemaphoreType.DMA,
        ],
        compiler_params=pltpu.CompilerParams(use_tc_tiling_on_sc=False, needs_layout_passes=False),
    )
    def k(dst_hbm, tbl_hbm, idxp_hbm, hist_hbm, poss_hbm,
          d_v, bank_v, tot_v, ps_v, pi_v, sem):
        wid = lax.axis_index("s") * 2 + lax.axis_index("c")
        base = wid * _EPW
        pltpu.sync_copy(dst_hbm.at[pl.ds(base, _EPW)], d_v)
        zero16 = jnp.zeros((16,), jnp.int32)

        def mz(i, c):
            bank_v[pl.ds(i * 16, 16)] = zero16
            return c

        lax.fori_loop(0, SPAD, mz, 0)
        lanes = lax.iota(jnp.int32, 16) * SPAD

        def acc(i, c):
            d = d_v[pl.ds(i * 16, 16)]
            addr = lanes + d
            old = plsc.load_gather(bank_v, [addr])
            plsc.store_scatter(bank_v, [addr], old + 1)
            return c

        lax.fori_loop(0, _NV, acc, 0)

        def red(j, c):
            sacc = jnp.zeros((16,), jnp.int32)
            for l in range(16):
                sacc = sacc + bank_v[pl.ds(l * SPAD + j * 16, 16)]
            tot_v[pl.ds(j * 16, 16)] = sacc
            return c

        lax.fori_loop(0, SPAD // 16, red, 0)
        pltpu.sync_copy(tot_v, hist_hbm.at[wid])

        sbase = wid * _SPW
        pltpu.sync_copy(idxp_hbm.at[pl.ds(sbase, _SPW)], pi_v)
        pltpu.async_copy(tbl_hbm.at[pi_v], ps_v, sem).wait()
        pltpu.sync_copy(ps_v, poss_hbm.at[pl.ds(sbase, _SPW), :])

    return k(dstp, tbl16, idxp)


def _sc_permute(dstp, srcp, tbase, tbl16, idxp):
    mesh = plsc.VectorSubcoreMesh(**_MESH)

    @functools.partial(
        pl.kernel,
        out_type=(
            jax.ShapeDtypeStruct((EPAD, 16), jnp.float32),
            jax.ShapeDtypeStruct((EPAD, 16), jnp.float32),
        ),
        mesh=mesh,
        scratch_types=[
            pltpu.VMEM((_EPW,), jnp.int32),
            pltpu.VMEM((_EPW,), jnp.int32),
            pltpu.VMEM((_EPW,), jnp.int32),
            pltpu.VMEM((_NCH, 128), jnp.int32),
            pltpu.VMEM((SPAD,), jnp.int32),
            pltpu.VMEM((SPAD,), jnp.int32),
            pltpu.VMEM((48,), jnp.int32),
            pltpu.VMEM((_EPW, 16), jnp.float32),
            pltpu.SemaphoreType.DMA,
        ],
        compiler_params=pltpu.CompilerParams(use_tc_tiling_on_sc=False, needs_layout_passes=False),
    )
    def k(dst_hbm, src_hbm, tb_hbm, tbl_hbm, idxp_hbm, a_hbm, b_hbm,
          d_v, s_v, c_v, slot_v, next_v, idx_v, sh_v, rows_v, sem):
        wid = lax.axis_index("s") * 2 + lax.axis_index("c")
        base = wid * _EPW
        pltpu.sync_copy(dst_hbm.at[pl.ds(base, _EPW)], d_v)
        pltpu.sync_copy(src_hbm.at[pl.ds(base, _EPW)], s_v)
        pltpu.sync_copy(tb_hbm.at[wid], next_v)
        pltpu.sync_copy(idxp_hbm, idx_v)
        neg16 = jnp.full((16,), -1, jnp.int32)
        sh_v[pl.ds(0, 16)] = neg16
        sh_v[pl.ds(32, 16)] = neg16
        pos = lax.iota(jnp.int32, 16)

        def vstep(i, c):
            d = d_v[pl.ds(i * 16, 16)]
            c_v[pl.ds(i * 16, 16)] = plsc.load_gather(idx_v, [d])
            sd, sl = plsc.sort_key_val(d, pos)
            sh_v[pl.ds(16, 16)] = sd
            prev = sh_v[pl.ds(15, 16)]
            nxt = sh_v[pl.ds(17, 16)]
            head = sd != prev
            tail = sd != nxt
            hp = plsc.cummax(jnp.where(head, pos, 0))
            r = pos - hp
            old = plsc.load_gather(next_v, [sd])
            plsc.store_scatter(next_v, [sd], old + r + 1, mask=tail)
            ch = i // 8
            kk = i % 8
            plsc.store_scatter(
                slot_v, [jnp.full((16,), ch, jnp.int32), kk * 16 + sl],
                old + r)
            return c

        lax.fori_loop(0, _NV, vstep, 0)

        def pump(ids_v, out_hbm):
            def fireg(ci, c):
                off = ci * 128
                pltpu.async_copy(tbl_hbm.at[ids_v.at[pl.ds(off, 128)]],
                                 rows_v.at[pl.ds(off, 128), :], sem)
                return c

            lax.fori_loop(0, _NCH, fireg, 0)
            pltpu.make_async_copy(
                tbl_hbm.at[pl.ds(0, _EPW), :], rows_v, sem).wait()

            def fires(ci, c):
                off = ci * 128
                pltpu.async_copy(rows_v.at[pl.ds(off, 128), :],
                                 out_hbm.at[slot_v.at[ci]], sem)
                return c

            lax.fori_loop(0, _NCH, fires, 0)
            pltpu.make_async_copy(
                out_hbm.at[pl.ds(0, _EPW), :], rows_v, sem).wait()

        pump(s_v, a_hbm)
        pump(c_v, b_hbm)

    return k(dstp, srcp, tbase, tbl16, idxp)


def _group_meta(poff8f, e0f, e1f, e2f):
    def body(po_ref, e0_ref, e1_ref, e2_ref, gs_ref, vc_ref):
        i = pl.program_id(0)
        po = po_ref[...]
        g8 = ((lax.broadcasted_iota(jnp.int32, (1, 256), 1)
               + i * 256) * 8).astype(jnp.float32)
        cmp = jnp.where(po <= g8, 1.0, 0.0)
        gseg = jnp.sum(cmp, axis=0, keepdims=True) - 1.0
        segid = lax.broadcasted_iota(jnp.int32, (SPAD, 1), 0).astype(jnp.float32)
        oh = jnp.where(segid == gseg, 1.0, 0.0)
        d0 = jnp.dot(e0_ref[...], oh, preferred_element_type=jnp.float32)
        d1 = jnp.dot(e1_ref[...], oh, preferred_element_type=jnp.float32)
        d2 = jnp.dot(e2_ref[...], oh, preferred_element_type=jnp.float32)
        endsel = (d2 * 128.0 + d1) * 128.0 + d0
        gs_ref[...] = gseg.astype(jnp.int32)[None]
        vc_ref[...] = jnp.clip(endsel - g8, 0.0, 8.0)[None]

    ngb = NG // 256
    return pl.pallas_call(
        body,
        grid=(ngb,),
        in_specs=[
            pl.BlockSpec((SPAD, 1), lambda i: (0, 0)),
            pl.BlockSpec((1, SPAD), lambda i: (0, 0)),
            pl.BlockSpec((1, SPAD), lambda i: (0, 0)),
            pl.BlockSpec((1, SPAD), lambda i: (0, 0)),
        ],
        out_specs=(
            pl.BlockSpec((1, 1, 256), lambda i: (i, 0, 0)),
            pl.BlockSpec((1, 1, 256), lambda i: (i, 0, 0)),
        ),
        out_shape=(
            jax.ShapeDtypeStruct((ngb, 1, 256), jnp.int32),
            jax.ShapeDtypeStruct((ngb, 1, 256), jnp.float32),
        ),
    )(poff8f, e0f, e1f, e2f)


BROW = 256
BGRP = BROW // 8


def _mlp_segmax(a, b, vrow, gkeys, w1, b1, w2, b2, w3, b3):
    def body(keys_ref, a_ref, b_ref, v_ref, w1_ref, b1_ref, w2_ref, b2_ref,
             w3_ref, b3_ref, out_ref, gm_ref):
        i = pl.program_id(0)

        @pl.when(i == 0)
        def _init():
            out_ref[...] = jnp.full((SPAD, 512), -jnp.inf, jnp.float32)

        msg = a_ref[...] - b_ref[...]
        h = jnp.dot(msg, w1_ref[...], preferred_element_type=jnp.float32)
        h = h + b1_ref[...]
        h = jnp.where(h > 0, h, 0.2 * h)
        h = jnp.dot(h, w2_ref[...], preferred_element_type=jnp.float32)
        h = h + b2_ref[...]
        h = jnp.where(h > 0, h, 0.2 * h)
        z = jnp.dot(h, w3_ref[...], preferred_element_type=jnp.float32)
        z = z + b3_ref[...]
        z = jnp.where(z > 0, z, 0.2 * z)
        z = jnp.where(v_ref[...] > 0, z, -jnp.inf)
        gm_ref[...] = jnp.max(z.reshape(BGRP, 8, 512), axis=1)

        def upd(j, carry):
            d = keys_ref[0, 0, j]
            row = gm_ref[pl.ds(j, 1), :]
            out_ref[pl.ds(d, 1), :] = jnp.maximum(out_ref[pl.ds(d, 1), :], row)
            return carry

        lax.fori_loop(0, BGRP, upd, 0)

    return pl.pallas_call(
        body,
        grid=(EPAD // BROW,),
        in_specs=[
            pl.BlockSpec((1, 1, BGRP), lambda i: (i, 0, 0),
                         memory_space=pltpu.SMEM),
            pl.BlockSpec((BROW, 16), lambda i: (i, 0)),
            pl.BlockSpec((BROW, 16), lambda i: (i, 0)),
            pl.BlockSpec((BROW, 1), lambda i: (i, 0)),
            pl.BlockSpec((16, 64), lambda i: (0, 0)),
            pl.BlockSpec((1, 64), lambda i: (0, 0)),
            pl.BlockSpec((64, 128), lambda i: (0, 0)),
            pl.BlockSpec((1, 128), lambda i: (0, 0)),
            pl.BlockSpec((128, 512), lambda i: (0, 0)),
            pl.BlockSpec((1, 512), lambda i: (0, 0)),
        ],
        out_specs=pl.BlockSpec((SPAD, 512), lambda i: (0, 0)),
        out_shape=jax.ShapeDtypeStruct((SPAD, 512), jnp.float32),
        scratch_shapes=[pltpu.VMEM((BGRP, 512), jnp.float32)],
        compiler_params=pltpu.CompilerParams(
            dimension_semantics=("arbitrary",)),
    )(gkeys, a, b, vrow, w1, b1, w2, b2, w3, b3)


def _encode_select(aggraw, poss, ew1a, ew1b, eb1, ew2, eb2):
    def body(agg_ref, poss_ref, w1a_ref, w1b_ref, b1_ref, w2_ref, b2_ref,
             z_ref, mi_ref, is_ref, sb_ref):
        raw = agg_ref[...]
        agg = jnp.where(raw < -1e30, 0.0, raw)
        p16 = poss_ref[...]
        h2 = (jnp.dot(agg, w1a_ref[...], preferred_element_type=jnp.float32)
              + jnp.dot(p16, w1b_ref[...], preferred_element_type=jnp.float32)
              + b1_ref[...])
        h2 = jnp.where(h2 > 0, h2, 0.2 * h2)
        out = jnp.dot(h2, w2_ref[...], preferred_element_type=jnp.float32)
        out = out + b2_ref[...]
        mean = out[:, :512]
        lv = out[:, 512:]
        invstd = jnp.exp(-0.5 * lv)
        std = jnp.exp(0.5 * lv)
        mi_ref[...] = mean * invstd
        is_ref[...] = invstd
        score = jnp.mean(std, axis=1, keepdims=True)
        rowid = lax.broadcasted_iota(jnp.int32, (SPAD, 1), 0)
        bcol = p16[:, 3:4]
        valid = rowid < S
        for bb in range(B):
            act0 = jnp.logical_and(valid, bcol == float(bb))
            sb_ref[...] = jnp.where(act0, score, jnp.inf)

            def rnd(r, carry):
                nu, de = carry
                cur = sb_ref[...]
                m = jnp.min(cur)
                cand = jnp.where(cur == m, rowid, jnp.int32(2 ** 30))
                isel = jnp.min(cand)
                nu = nu + mi_ref[pl.ds(isel, 1), :]
                de = de + is_ref[pl.ds(isel, 1), :]
                sb_ref[pl.ds(isel, 1), :] = jnp.full((1, 1), jnp.inf,
                                                     jnp.float32)
                return nu, de

            nu, de = lax.fori_loop(
                0, NC, rnd,
                (jnp.zeros((1, 512), jnp.float32),
                 jnp.zeros((1, 512), jnp.float32)))
            z_ref[bb:bb + 1, :] = nu / de

    return pl.pallas_call(
        body,
        grid=(1,),
        in_specs=[
            pl.BlockSpec((SPAD, 512), lambda i: (0, 0)),
            pl.BlockSpec((SPAD, 16), lambda i: (0, 0)),
            pl.BlockSpec((512, 512), lambda i: (0, 0)),
            pl.BlockSpec((16, 512), lambda i: (0, 0)),
            pl.BlockSpec((1, 512), lambda i: (0, 0)),
            pl.BlockSpec((512, 1024), lambda i: (0, 0)),
            pl.BlockSpec((1, 1024), lambda i: (0, 0)),
        ],
        out_specs=pl.BlockSpec((8, 512), lambda i: (0, 0)),
        out_shape=jax.ShapeDtypeStruct((8, 512), jnp.float32),
        scratch_shapes=[
            pltpu.VMEM((SPAD, 512), jnp.float32),
            pltpu.VMEM((SPAD, 512), jnp.float32),
            pltpu.VMEM((SPAD, 1), jnp.float32),
        ],
        compiler_params=pltpu.CompilerParams(
            vmem_limit_bytes=100 * 1024 * 1024),
    )(aggraw, poss, ew1a, ew1b, eb1, ew2, eb2)


def _decode(z, gg, f1w1a, f1w1g, f1b1, f1w2, f1b2, f1w3p, f1b3p,
            f2w1a, f2w1b, f2b1, f2w2, f2b2, f2w3p, f2b3p):
    def body(z_ref, gg_ref, w1a, w1g, bb1, w12, b12, w13, b13,
             w2a, w2b, bb2, w22, b22, w23, b23, out_ref):
        zf = z_ref[...]
        gg_ = gg_ref[...]
        bcol = gg_[:, 3:4]
        code = jnp.zeros((256, 512), jnp.float32)
        for bb in range(B):
            code = jnp.where(bcol == float(bb), zf[bb:bb + 1, :], code)
        h = (jnp.dot(code, w1a[...], preferred_element_type=jnp.float32)
             + jnp.dot(gg_, w1g[...], preferred_element_type=jnp.float32)
             + bb1[...])
        h = jnp.maximum(h, 0.0)
        h = jnp.dot(h, w12[...], preferred_element_type=jnp.float32) + b12[...]
        h = jnp.maximum(h, 0.0)
        x1 = jnp.dot(h, w13[...], preferred_element_type=jnp.float32) + b13[...]
        h = (jnp.dot(code, w2a[...], preferred_element_type=jnp.float32)
             + jnp.dot(x1, w2b[...], preferred_element_type=jnp.float32)
             + bb2[...])
        h = jnp.maximum(h, 0.0)
        h = jnp.dot(h, w22[...], preferred_element_type=jnp.float32) + b22[...]
        h = jnp.maximum(h, 0.0)
        out_ref[...] = (jnp.dot(h, w23[...], preferred_element_type=jnp.float32)
                        + b23[...])

    wspec = lambda r, c: pl.BlockSpec((r, c), lambda i: (0, 0))
    return pl.pallas_call(
        body,
        grid=(DBLK,),
        in_specs=[
            pl.BlockSpec((8, 512), lambda i: (0, 0)),
            pl.BlockSpec((256, 16), lambda i: (i, 0)),
            wspec(512, 512), wspec(16, 512), wspec(1, 512),
            wspec(512, 512), wspec(1, 512),
            wspec(512, 16), wspec(1, 16),
            wspec(512, 512), wspec(16, 512), wspec(1, 512),
            wspec(512, 512), wspec(1, 512),
            wspec(512, 16), wspec(1, 16),
        ],
        out_specs=pl.BlockSpec((256, 16), lambda i: (i, 0)),
        out_shape=jax.ShapeDtypeStruct((DPAD, 16), jnp.float32),
        compiler_params=pltpu.CompilerParams(
            dimension_semantics=("arbitrary",)),
    )(z, gg, f1w1a, f1w1g, f1b1, f1w2, f1b2, f1w3p, f1b3p,
      f2w1a, f2w1b, f2b1, f2w2, f2b2, f2w3p, f2b3p)


def _np_grid16():
    g1, g2 = np.meshgrid(np.linspace(-0.3, 0.3, 45), np.linspace(-0.3, 0.3, 45))
    g = np.stack([g1.reshape(-1), g2.reshape(-1)], axis=-1).astype(np.float32)
    gg = np.zeros((DPAD, 16), np.float32)
    t = np.arange(DPAD)
    bcol = np.minimum(t // G, B - 1)
    gg[:B * G, 0:2] = np.tile(g, (B, 1))
    gg[:, 3] = bcol
    return gg


_GG16_NP = _np_grid16()


def kernel(pos, batch, idx, src, dst, lW1, lb1, lW2, lb2, lW3, lb3,
           eW1, eb1, eW2, eb2,
           f1W1, f1b1, f1W2, f1b2, f1W3, f1b3,
           f2W1, f2b1, f2W2, f2b2, f2W3, f2b3):
    i32 = jnp.int32
    dstp = jnp.concatenate([dst.astype(i32),
                            jnp.full((EP - E,), SPAD - 1, i32)])
    srcp = jnp.concatenate([src.astype(i32), jnp.zeros((EP - E,), i32)])
    idxp = jnp.concatenate([idx.astype(i32), jnp.zeros((SPAD - S,), i32)])
    tbl16 = jnp.concatenate(
        [pos, batch.astype(jnp.float32)[:, None],
         jnp.zeros((N, 12), jnp.float32)], axis=1)

    hists, poss = _sc_hist(dstp, tbl16, idxp)

    counts = jnp.sum(hists, axis=0)
    pc8 = (counts + 7) // 8 * 8
    csum = jnp.cumsum(pc8)
    poff8 = jnp.concatenate([jnp.zeros((1,), i32), csum[:-1]])
    tbase = poff8[None, :] + (jnp.cumsum(hists, axis=0) - hists)
    poff8f = poff8.astype(jnp.float32).reshape(SPAD, 1)
    endv = poff8 + counts
    e0f = (endv % 128).astype(jnp.float32).reshape(1, SPAD)
    e1f = (endv // 128 % 128).astype(jnp.float32).reshape(1, SPAD)
    e2f = (endv // 16384).astype(jnp.float32).reshape(1, SPAD)
    gsegb, vcntb = _group_meta(poff8f, e0f, e1f, e2f)
    vrow = ((jnp.arange(8, dtype=jnp.float32)[None, :]
             < vcntb.reshape(NG, 1))
            .astype(jnp.float32).reshape(EPAD, 1))
    gkeys = gsegb.reshape(EPAD // BROW, 1, BGRP)

    a, b = _sc_permute(dstp, srcp, tbase, tbl16, idxp)

    w1 = jnp.zeros((16, 64), jnp.float32).at[:3].set(lW1)
    aggraw = _mlp_segmax(a, b, vrow, gkeys, w1, lb1[None, :], lW2,
                         lb2[None, :], lW3, lb3[None, :])

    ew1a = eW1[:512]
    ew1b = jnp.zeros((16, 512), jnp.float32).at[:3].set(eW1[512:515])
    z = _encode_select(aggraw, poss, ew1a, ew1b, eb1[None, :], eW2,
                       eb2[None, :])

    f1w1a = f1W1[:512]
    f1w1g = jnp.zeros((16, 512), jnp.float32).at[:2].set(f1W1[512:514])
    f1w3p = jnp.zeros((512, 16), jnp.float32).at[:, :3].set(f1W3)
    f1b3p = jnp.zeros((1, 16), jnp.float32).at[0, :3].set(f1b3)
    f2w1a = f2W1[:512]
    f2w1b = jnp.zeros((16, 512), jnp.float32).at[:3].set(f2W1[512:515])
    f2w3p = jnp.zeros((512, 16), jnp.float32).at[:, :3].set(f2W3)
    f2b3p = jnp.zeros((1, 16), jnp.float32).at[0, :3].set(f2b3)
    outp = _decode(z, jnp.asarray(_GG16_NP), f1w1a, f1w1g, f1b1[None, :],
                   f1W2, f1b2[None, :], f1w3p, f1b3p, f2w1a, f2w1b,
                   f2b1[None, :], f2W2, f2b2[None, :], f2w3p, f2b3p)
    return outp[:B * G, :3].reshape(B, G, 3)


# speedup vs baseline: 26.0516x; 1.2892x over previous
import functools

import jax
import jax.numpy as jnp
import numpy as np
from jax import lax
from jax.experimental import pallas as pl
from jax.experimental.pallas import tpu as pltpu
from jax.experimental.pallas import tpu_sc as plsc

N = 10000
B = 8
S = 2000
E = 160000
K = 512
NC = 10
G = 45 * 45

SPAD = 2048
EP = 163840
EPAD = 178176
NBLK = EPAD // 256
NG = EPAD // 8
DPAD = 16384
DBLK = DPAD // 256

_NWORK = 32
_EPW = EP // _NWORK
_NV = _EPW // 16
_NCH = _EPW // 128
_SPW = SPAD // _NWORK

_MESH = dict(core_axis_name="c", subcore_axis_name="s")


def _sc_hist(dstp, tbl16, idxp):
    mesh = plsc.VectorSubcoreMesh(**_MESH)

    @functools.partial(
        pl.kernel,
        out_type=(
            jax.ShapeDtypeStruct((_NWORK, SPAD), jnp.int32),
            jax.ShapeDtypeStruct((SPAD, 16), jnp.float32),
        ),
        mesh=mesh,
        scratch_types=[
            pltpu.VMEM((_EPW,), jnp.int32),
            pltpu.VMEM((16 * SPAD,), jnp.int32),
            pltpu.VMEM((SPAD,), jnp.int32),
            pltpu.VMEM((_SPW, 16), jnp.float32),
            pltpu.VMEM((_SPW,), jnp.int32),
            pltpu.SemaphoreType.DMA,
        ],
        compiler_params=pltpu.CompilerParams(use_tc_tiling_on_sc=False, needs_layout_passes=False),
    )
    def k(dst_hbm, tbl_hbm, idxp_hbm, hist_hbm, poss_hbm,
          d_v, bank_v, tot_v, ps_v, pi_v, sem):
        wid = lax.axis_index("s") * 2 + lax.axis_index("c")
        base = wid * _EPW
        pltpu.sync_copy(dst_hbm.at[pl.ds(base, _EPW)], d_v)
        zero16 = jnp.zeros((16,), jnp.int32)

        def mz(i, c):
            bank_v[pl.ds(i * 16, 16)] = zero16
            return c

        lax.fori_loop(0, SPAD, mz, 0)
        lanes = lax.iota(jnp.int32, 16) * SPAD

        def acc(i, c):
            d = d_v[pl.ds(i * 16, 16)]
            addr = lanes + d
            old = plsc.load_gather(bank_v, [addr])
            plsc.store_scatter(bank_v, [addr], old + 1)
            return c

        lax.fori_loop(0, _NV, acc, 0)

        def red(j, c):
            sacc = jnp.zeros((16,), jnp.int32)
            for l in range(16):
                sacc = sacc + bank_v[pl.ds(l * SPAD + j * 16, 16)]
            tot_v[pl.ds(j * 16, 16)] = sacc
            return c

        lax.fori_loop(0, SPAD // 16, red, 0)
        pltpu.sync_copy(tot_v, hist_hbm.at[wid])

        sbase = wid * _SPW
        pltpu.sync_copy(idxp_hbm.at[pl.ds(sbase, _SPW)], pi_v)
        pltpu.async_copy(tbl_hbm.at[pi_v], ps_v, sem).wait()
        pltpu.sync_copy(ps_v, poss_hbm.at[pl.ds(sbase, _SPW), :])

    return k(dstp, tbl16, idxp)


def _sc_permute(dstp, srcp, tbase, tbl16, idxp):
    mesh = plsc.VectorSubcoreMesh(**_MESH)

    @functools.partial(
        pl.kernel,
        out_type=(
            jax.ShapeDtypeStruct((EPAD, 16), jnp.float32),
            jax.ShapeDtypeStruct((EPAD, 16), jnp.float32),
        ),
        mesh=mesh,
        scratch_types=[
            pltpu.VMEM((_EPW,), jnp.int32),
            pltpu.VMEM((_EPW,), jnp.int32),
            pltpu.VMEM((_EPW,), jnp.int32),
            pltpu.VMEM((_NCH, 128), jnp.int32),
            pltpu.VMEM((SPAD,), jnp.int32),
            pltpu.VMEM((SPAD,), jnp.int32),
            pltpu.VMEM((48,), jnp.int32),
            pltpu.VMEM((_EPW, 16), jnp.float32),
            pltpu.SemaphoreType.DMA,
        ],
        compiler_params=pltpu.CompilerParams(use_tc_tiling_on_sc=False, needs_layout_passes=False),
    )
    def k(dst_hbm, src_hbm, tb_hbm, tbl_hbm, idxp_hbm, a_hbm, b_hbm,
          d_v, s_v, c_v, slot_v, next_v, idx_v, sh_v, rows_v, sem):
        wid = lax.axis_index("s") * 2 + lax.axis_index("c")
        base = wid * _EPW
        pltpu.sync_copy(dst_hbm.at[pl.ds(base, _EPW)], d_v)
        pltpu.sync_copy(src_hbm.at[pl.ds(base, _EPW)], s_v)
        pltpu.sync_copy(tb_hbm.at[wid], next_v)
        pltpu.sync_copy(idxp_hbm, idx_v)
        neg16 = jnp.full((16,), -1, jnp.int32)
        sh_v[pl.ds(0, 16)] = neg16
        sh_v[pl.ds(32, 16)] = neg16
        pos = lax.iota(jnp.int32, 16)

        def vstep(i, c):
            d = d_v[pl.ds(i * 16, 16)]
            c_v[pl.ds(i * 16, 16)] = plsc.load_gather(idx_v, [d])
            sd, sl = plsc.sort_key_val(d, pos)
            sh_v[pl.ds(16, 16)] = sd
            prev = sh_v[pl.ds(15, 16)]
            nxt = sh_v[pl.ds(17, 16)]
            head = sd != prev
            tail = sd != nxt
            hp = plsc.cummax(jnp.where(head, pos, 0))
            r = pos - hp
            old = plsc.load_gather(next_v, [sd])
            plsc.store_scatter(next_v, [sd], old + r + 1, mask=tail)
            ch = i // 8
            kk = i % 8
            plsc.store_scatter(
                slot_v, [jnp.full((16,), ch, jnp.int32), kk * 16 + sl],
                old + r)
            return c

        lax.fori_loop(0, _NV, vstep, 0)

        def pump(ids_v, out_hbm):
            def fireg(ci, c):
                off = ci * 128
                pltpu.async_copy(tbl_hbm.at[ids_v.at[pl.ds(off, 128)]],
                                 rows_v.at[pl.ds(off, 128), :], sem)
                return c

            lax.fori_loop(0, _NCH, fireg, 0)
            pltpu.make_async_copy(
                tbl_hbm.at[pl.ds(0, _EPW), :], rows_v, sem).wait()

            def fires(ci, c):
                off = ci * 128
                pltpu.async_copy(rows_v.at[pl.ds(off, 128), :],
                                 out_hbm.at[slot_v.at[ci]], sem)
                return c

            lax.fori_loop(0, _NCH, fires, 0)
            pltpu.make_async_copy(
                out_hbm.at[pl.ds(0, _EPW), :], rows_v, sem).wait()

        pump(s_v, a_hbm)
        pump(c_v, b_hbm)

    return k(dstp, srcp, tbase, tbl16, idxp)


def _group_meta(poff8f, e0f, e1f, e2f):
    def body(po_ref, e0_ref, e1_ref, e2_ref, gs_ref, vc_ref):
        i = pl.program_id(0)
        po = po_ref[...]
        g8 = ((lax.broadcasted_iota(jnp.int32, (1, 256), 1)
               + i * 256) * 8).astype(jnp.float32)
        cmp = jnp.where(po <= g8, 1.0, 0.0)
        gseg = jnp.sum(cmp, axis=0, keepdims=True) - 1.0
        segid = lax.broadcasted_iota(jnp.int32, (SPAD, 1), 0).astype(jnp.float32)
        oh = jnp.where(segid == gseg, 1.0, 0.0)
        d0 = jnp.dot(e0_ref[...], oh, preferred_element_type=jnp.float32)
        d1 = jnp.dot(e1_ref[...], oh, preferred_element_type=jnp.float32)
        d2 = jnp.dot(e2_ref[...], oh, preferred_element_type=jnp.float32)
        endsel = (d2 * 128.0 + d1) * 128.0 + d0
        gs_ref[...] = gseg.astype(jnp.int32)[None]
        vc_ref[...] = jnp.clip(endsel - g8, 0.0, 8.0)[None]

    ngb = NG // 256
    return pl.pallas_call(
        body,
        grid=(ngb,),
        in_specs=[
            pl.BlockSpec((SPAD, 1), lambda i: (0, 0)),
            pl.BlockSpec((1, SPAD), lambda i: (0, 0)),
            pl.BlockSpec((1, SPAD), lambda i: (0, 0)),
            pl.BlockSpec((1, SPAD), lambda i: (0, 0)),
        ],
        out_specs=(
            pl.BlockSpec((1, 1, 256), lambda i: (i, 0, 0)),
            pl.BlockSpec((1, 1, 256), lambda i: (i, 0, 0)),
        ),
        out_shape=(
            jax.ShapeDtypeStruct((ngb, 1, 256), jnp.int32),
            jax.ShapeDtypeStruct((ngb, 1, 256), jnp.float32),
        ),
    )(poff8f, e0f, e1f, e2f)


BROW = 1024
BGRP = BROW // 8


def _mlp_segmax(a, b, vrow, gkeys, w1, b1, w2, b2, w3, b3):
    def body(keys_ref, a_ref, b_ref, v_ref, w1_ref, b1_ref, w2_ref, b2_ref,
             w3_ref, b3_ref, out_ref, gm_ref):
        i = pl.program_id(0)

        @pl.when(i == 0)
        def _init():
            out_ref[...] = jnp.full((SPAD, 512), -jnp.inf, jnp.float32)

        msg = a_ref[...] - b_ref[...]
        h = jnp.dot(msg, w1_ref[...], preferred_element_type=jnp.float32)
        h = h + b1_ref[...]
        h = jnp.where(h > 0, h, 0.2 * h)
        h = jnp.dot(h, w2_ref[...], preferred_element_type=jnp.float32)
        h = h + b2_ref[...]
        h = jnp.where(h > 0, h, 0.2 * h)
        z = jnp.dot(h, w3_ref[...], preferred_element_type=jnp.float32)
        z = z + b3_ref[...]
        z = jnp.where(z > 0, z, 0.2 * z)
        z = jnp.where(v_ref[...] > 0, z, -jnp.inf)
        gm_ref[...] = jnp.max(z.reshape(BGRP, 8, 512), axis=1)

        def upd(j, carry):
            d = keys_ref[0, 0, j]
            row = gm_ref[pl.ds(j, 1), :]
            out_ref[pl.ds(d, 1), :] = jnp.maximum(out_ref[pl.ds(d, 1), :], row)
            return carry

        lax.fori_loop(0, BGRP, upd, 0)

    return pl.pallas_call(
        body,
        grid=(EPAD // BROW,),
        in_specs=[
            pl.BlockSpec((1, 1, BGRP), lambda i: (i, 0, 0),
                         memory_space=pltpu.SMEM),
            pl.BlockSpec((BROW, 16), lambda i: (i, 0)),
            pl.BlockSpec((BROW, 16), lambda i: (i, 0)),
            pl.BlockSpec((BROW, 1), lambda i: (i, 0)),
            pl.BlockSpec((16, 64), lambda i: (0, 0)),
            pl.BlockSpec((1, 64), lambda i: (0, 0)),
            pl.BlockSpec((64, 128), lambda i: (0, 0)),
            pl.BlockSpec((1, 128), lambda i: (0, 0)),
            pl.BlockSpec((128, 512), lambda i: (0, 0)),
            pl.BlockSpec((1, 512), lambda i: (0, 0)),
        ],
        out_specs=pl.BlockSpec((SPAD, 512), lambda i: (0, 0)),
        out_shape=jax.ShapeDtypeStruct((SPAD, 512), jnp.float32),
        scratch_shapes=[pltpu.VMEM((BGRP, 512), jnp.float32)],
        compiler_params=pltpu.CompilerParams(
            dimension_semantics=("arbitrary",)),
    )(gkeys, a, b, vrow, w1, b1, w2, b2, w3, b3)


def _encode_select(aggraw, poss, ew1a, ew1b, eb1, ew2, eb2):
    def body(agg_ref, poss_ref, w1a_ref, w1b_ref, b1_ref, w2_ref, b2_ref,
             z_ref, mi_ref, is_ref, sb_ref):
        raw = agg_ref[...]
        agg = jnp.where(raw < -1e30, 0.0, raw)
        p16 = poss_ref[...]
        h2 = (jnp.dot(agg, w1a_ref[...], preferred_element_type=jnp.float32)
              + jnp.dot(p16, w1b_ref[...], preferred_element_type=jnp.float32)
              + b1_ref[...])
        h2 = jnp.where(h2 > 0, h2, 0.2 * h2)
        out = jnp.dot(h2, w2_ref[...], preferred_element_type=jnp.float32)
        out = out + b2_ref[...]
        mean = out[:, :512]
        lv = out[:, 512:]
        invstd = jnp.exp(-0.5 * lv)
        std = jnp.exp(0.5 * lv)
        mi_ref[...] = mean * invstd
        is_ref[...] = invstd
        score = jnp.mean(std, axis=1, keepdims=True)
        rowid = lax.broadcasted_iota(jnp.int32, (SPAD, 1), 0)
        bcol = p16[:, 3:4]
        valid = rowid < S
        for bb in range(B):
            act0 = jnp.logical_and(valid, bcol == float(bb))
            sb_ref[...] = jnp.where(act0, score, jnp.inf)

            def rnd(r, carry):
                nu, de = carry
                cur = sb_ref[...]
                m = jnp.min(cur)
                cand = jnp.where(cur == m, rowid, jnp.int32(2 ** 30))
                isel = jnp.min(cand)
                nu = nu + mi_ref[pl.ds(isel, 1), :]
                de = de + is_ref[pl.ds(isel, 1), :]
                sb_ref[pl.ds(isel, 1), :] = jnp.full((1, 1), jnp.inf,
                                                     jnp.float32)
                return nu, de

            nu, de = lax.fori_loop(
                0, NC, rnd,
                (jnp.zeros((1, 512), jnp.float32),
                 jnp.zeros((1, 512), jnp.float32)))
            z_ref[bb:bb + 1, :] = nu / de

    return pl.pallas_call(
        body,
        grid=(1,),
        in_specs=[
            pl.BlockSpec((SPAD, 512), lambda i: (0, 0)),
            pl.BlockSpec((SPAD, 16), lambda i: (0, 0)),
            pl.BlockSpec((512, 512), lambda i: (0, 0)),
            pl.BlockSpec((16, 512), lambda i: (0, 0)),
            pl.BlockSpec((1, 512), lambda i: (0, 0)),
            pl.BlockSpec((512, 1024), lambda i: (0, 0)),
            pl.BlockSpec((1, 1024), lambda i: (0, 0)),
        ],
        out_specs=pl.BlockSpec((8, 512), lambda i: (0, 0)),
        out_shape=jax.ShapeDtypeStruct((8, 512), jnp.float32),
        scratch_shapes=[
            pltpu.VMEM((SPAD, 512), jnp.float32),
            pltpu.VMEM((SPAD, 512), jnp.float32),
            pltpu.VMEM((SPAD, 1), jnp.float32),
        ],
        compiler_params=pltpu.CompilerParams(
            vmem_limit_bytes=100 * 1024 * 1024),
    )(aggraw, poss, ew1a, ew1b, eb1, ew2, eb2)


def _decode(z, gg, f1w1a, f1w1g, f1b1, f1w2, f1b2, f1w3p, f1b3p,
            f2w1a, f2w1b, f2b1, f2w2, f2b2, f2w3p, f2b3p):
    def body(z_ref, gg_ref, w1a, w1g, bb1, w12, b12, w13, b13,
             w2a, w2b, bb2, w22, b22, w23, b23, out_ref):
        zf = z_ref[...]
        gg_ = gg_ref[...]
        bcol = gg_[:, 3:4]
        code = jnp.zeros((256, 512), jnp.float32)
        for bb in range(B):
            code = jnp.where(bcol == float(bb), zf[bb:bb + 1, :], code)
        h = (jnp.dot(code, w1a[...], preferred_element_type=jnp.float32)
             + jnp.dot(gg_, w1g[...], preferred_element_type=jnp.float32)
             + bb1[...])
        h = jnp.maximum(h, 0.0)
        h = jnp.dot(h, w12[...], preferred_element_type=jnp.float32) + b12[...]
        h = jnp.maximum(h, 0.0)
        x1 = jnp.dot(h, w13[...], preferred_element_type=jnp.float32) + b13[...]
        h = (jnp.dot(code, w2a[...], preferred_element_type=jnp.float32)
             + jnp.dot(x1, w2b[...], preferred_element_type=jnp.float32)
             + bb2[...])
        h = jnp.maximum(h, 0.0)
        h = jnp.dot(h, w22[...], preferred_element_type=jnp.float32) + b22[...]
        h = jnp.maximum(h, 0.0)
        out_ref[...] = (jnp.dot(h, w23[...], preferred_element_type=jnp.float32)
                        + b23[...])

    wspec = lambda r, c: pl.BlockSpec((r, c), lambda i: (0, 0))
    return pl.pallas_call(
        body,
        grid=(DBLK,),
        in_specs=[
            pl.BlockSpec((8, 512), lambda i: (0, 0)),
            pl.BlockSpec((256, 16), lambda i: (i, 0)),
            wspec(512, 512), wspec(16, 512), wspec(1, 512),
            wspec(512, 512), wspec(1, 512),
            wspec(512, 16), wspec(1, 16),
            wspec(512, 512), wspec(16, 512), wspec(1, 512),
            wspec(512, 512), wspec(1, 512),
            wspec(512, 16), wspec(1, 16),
        ],
        out_specs=pl.BlockSpec((256, 16), lambda i: (i, 0)),
        out_shape=jax.ShapeDtypeStruct((DPAD, 16), jnp.float32),
        compiler_params=pltpu.CompilerParams(
            dimension_semantics=("arbitrary",)),
    )(z, gg, f1w1a, f1w1g, f1b1, f1w2, f1b2, f1w3p, f1b3p,
      f2w1a, f2w1b, f2b1, f2w2, f2b2, f2w3p, f2b3p)


def _np_grid16():
    g1, g2 = np.meshgrid(np.linspace(-0.3, 0.3, 45), np.linspace(-0.3, 0.3, 45))
    g = np.stack([g1.reshape(-1), g2.reshape(-1)], axis=-1).astype(np.float32)
    gg = np.zeros((DPAD, 16), np.float32)
    t = np.arange(DPAD)
    bcol = np.minimum(t // G, B - 1)
    gg[:B * G, 0:2] = np.tile(g, (B, 1))
    gg[:, 3] = bcol
    return gg


_GG16_NP = _np_grid16()


def kernel(pos, batch, idx, src, dst, lW1, lb1, lW2, lb2, lW3, lb3,
           eW1, eb1, eW2, eb2,
           f1W1, f1b1, f1W2, f1b2, f1W3, f1b3,
           f2W1, f2b1, f2W2, f2b2, f2W3, f2b3):
    i32 = jnp.int32
    dstp = jnp.concatenate([dst.astype(i32),
                            jnp.full((EP - E,), SPAD - 1, i32)])
    srcp = jnp.concatenate([src.astype(i32), jnp.zeros((EP - E,), i32)])
    idxp = jnp.concatenate([idx.astype(i32), jnp.zeros((SPAD - S,), i32)])
    tbl16 = jnp.concatenate(
        [pos, batch.astype(jnp.float32)[:, None],
         jnp.zeros((N, 12), jnp.float32)], axis=1)

    hists, poss = _sc_hist(dstp, tbl16, idxp)

    counts = jnp.sum(hists, axis=0)
    pc8 = (counts + 7) // 8 * 8
    csum = jnp.cumsum(pc8)
    poff8 = jnp.concatenate([jnp.zeros((1,), i32), csum[:-1]])
    tbase = poff8[None, :] + (jnp.cumsum(hists, axis=0) - hists)
    poff8f = poff8.astype(jnp.float32).reshape(SPAD, 1)
    endv = poff8 + counts
    e0f = (endv % 128).astype(jnp.float32).reshape(1, SPAD)
    e1f = (endv // 128 % 128).astype(jnp.float32).reshape(1, SPAD)
    e2f = (endv // 16384).astype(jnp.float32).reshape(1, SPAD)
    gsegb, vcntb = _group_meta(poff8f, e0f, e1f, e2f)
    vrow = ((jnp.arange(8, dtype=jnp.float32)[None, :]
             < vcntb.reshape(NG, 1))
            .astype(jnp.float32).reshape(EPAD, 1))
    gkeys = gsegb.reshape(EPAD // BROW, 1, BGRP)

    a, b = _sc_permute(dstp, srcp, tbase, tbl16, idxp)

    w1 = jnp.zeros((16, 64), jnp.float32).at[:3].set(lW1)
    aggraw = _mlp_segmax(a, b, vrow, gkeys, w1, lb1[None, :], lW2,
                         lb2[None, :], lW3, lb3[None, :])

    ew1a = eW1[:512]
    ew1b = jnp.zeros((16, 512), jnp.float32).at[:3].set(eW1[512:515])
    z = _encode_select(aggraw, poss, ew1a, ew1b, eb1[None, :], eW2,
                       eb2[None, :])

    f1w1a = f1W1[:512]
    f1w1g = jnp.zeros((16, 512), jnp.float32).at[:2].set(f1W1[512:514])
    f1w3p = jnp.zeros((512, 16), jnp.float32).at[:, :3].set(f1W3)
    f1b3p = jnp.zeros((1, 16), jnp.float32).at[0, :3].set(f1b3)
    f2w1a = f2W1[:512]
    f2w1b = jnp.zeros((16, 512), jnp.float32).at[:3].set(f2W1[512:515])
    f2w3p = jnp.zeros((512, 16), jnp.float32).at[:, :3].set(f2W3)
    f2b3p = jnp.zeros((1, 16), jnp.float32).at[0, :3].set(f2b3)
    outp = _decode(z, jnp.asarray(_GG16_NP), f1w1a, f1w1g, f1b1[None, :],
                   f1W2, f1b2[None, :], f1w3p, f1b3p, f2w1a, f2w1b,
                   f2b1[None, :], f2W2, f2b2[None, :], f2w3p, f2b3p)
    return outp[:B * G, :3].reshape(B, G, 3)


# speedup vs baseline: 26.9246x; 1.0335x over previous
import functools

import jax
import jax.numpy as jnp
import numpy as np
from jax import lax
from jax.experimental import pallas as pl
from jax.experimental.pallas import tpu as pltpu
from jax.experimental.pallas import tpu_sc as plsc

N = 10000
B = 8
S = 2000
E = 160000
K = 512
NC = 10
G = 45 * 45

SPAD = 2048
EP = 163840
EPAD = 178176
NBLK = EPAD // 256
NG = EPAD // 8
DPAD = 16384
DBLK = DPAD // 256

_NWORK = 32
_EPW = EP // _NWORK
_NV = _EPW // 16
_NCH = _EPW // 128
_SPW = SPAD // _NWORK

_MESH = dict(core_axis_name="c", subcore_axis_name="s")


def _sc_hist(dstp, tbl16, idxp):
    mesh = plsc.VectorSubcoreMesh(**_MESH)

    @functools.partial(
        pl.kernel,
        out_type=(
            jax.ShapeDtypeStruct((_NWORK, SPAD), jnp.int32),
            jax.ShapeDtypeStruct((SPAD, 16), jnp.float32),
        ),
        mesh=mesh,
        scratch_types=[
            pltpu.VMEM((_EPW,), jnp.int32),
            pltpu.VMEM((16 * SPAD,), jnp.int32),
            pltpu.VMEM((SPAD,), jnp.int32),
            pltpu.VMEM((_SPW, 16), jnp.float32),
            pltpu.VMEM((_SPW,), jnp.int32),
            pltpu.SemaphoreType.DMA,
        ],
        compiler_params=pltpu.CompilerParams(use_tc_tiling_on_sc=False, needs_layout_passes=False),
    )
    def k(dst_hbm, tbl_hbm, idxp_hbm, hist_hbm, poss_hbm,
          d_v, bank_v, tot_v, ps_v, pi_v, sem):
        wid = lax.axis_index("s") * 2 + lax.axis_index("c")
        base = wid * _EPW
        pltpu.sync_copy(dst_hbm.at[pl.ds(base, _EPW)], d_v)
        zero16 = jnp.zeros((16,), jnp.int32)

        def mz(i, c):
            bank_v[pl.ds(i * 16, 16)] = zero16
            return c

        lax.fori_loop(0, SPAD, mz, 0)
        lanes = lax.iota(jnp.int32, 16) * SPAD

        def acc(i, c):
            d = d_v[pl.ds(i * 16, 16)]
            addr = lanes + d
            old = plsc.load_gather(bank_v, [addr])
            plsc.store_scatter(bank_v, [addr], old + 1)
            return c

        lax.fori_loop(0, _NV, acc, 0)

        def red(j, c):
            sacc = jnp.zeros((16,), jnp.int32)
            for l in range(16):
                sacc = sacc + bank_v[pl.ds(l * SPAD + j * 16, 16)]
            tot_v[pl.ds(j * 16, 16)] = sacc
            return c

        lax.fori_loop(0, SPAD // 16, red, 0)
        pltpu.sync_copy(tot_v, hist_hbm.at[wid])

        sbase = wid * _SPW
        pltpu.sync_copy(idxp_hbm.at[pl.ds(sbase, _SPW)], pi_v)
        pltpu.async_copy(tbl_hbm.at[pi_v], ps_v, sem).wait()
        pltpu.sync_copy(ps_v, poss_hbm.at[pl.ds(sbase, _SPW), :])

    return k(dstp, tbl16, idxp)


def _sc_permute(dstp, srcp, tbase, tbl16, idxp):
    mesh = plsc.VectorSubcoreMesh(**_MESH)

    @functools.partial(
        pl.kernel,
        out_type=(
            jax.ShapeDtypeStruct((EPAD, 16), jnp.float32),
            jax.ShapeDtypeStruct((EPAD, 16), jnp.float32),
        ),
        mesh=mesh,
        scratch_types=[
            pltpu.VMEM((_EPW,), jnp.int32),
            pltpu.VMEM((_EPW,), jnp.int32),
            pltpu.VMEM((_EPW,), jnp.int32),
            pltpu.VMEM((_NCH, 128), jnp.int32),
            pltpu.VMEM((SPAD,), jnp.int32),
            pltpu.VMEM((SPAD,), jnp.int32),
            pltpu.VMEM((48,), jnp.int32),
            pltpu.VMEM((_EPW, 16), jnp.float32),
            pltpu.SemaphoreType.DMA,
        ],
        compiler_params=pltpu.CompilerParams(use_tc_tiling_on_sc=False, needs_layout_passes=False),
    )
    def k(dst_hbm, src_hbm, tb_hbm, tbl_hbm, idxp_hbm, a_hbm, b_hbm,
          d_v, s_v, c_v, slot_v, next_v, idx_v, sh_v, rows_v, sem):
        wid = lax.axis_index("s") * 2 + lax.axis_index("c")
        base = wid * _EPW
        pltpu.sync_copy(dst_hbm.at[pl.ds(base, _EPW)], d_v)
        pltpu.sync_copy(src_hbm.at[pl.ds(base, _EPW)], s_v)
        pltpu.sync_copy(tb_hbm.at[wid], next_v)
        pltpu.sync_copy(idxp_hbm, idx_v)
        neg16 = jnp.full((16,), -1, jnp.int32)
        sh_v[pl.ds(0, 16)] = neg16
        sh_v[pl.ds(32, 16)] = neg16
        pos = lax.iota(jnp.int32, 16)

        def vstep(i, c):
            d = d_v[pl.ds(i * 16, 16)]
            c_v[pl.ds(i * 16, 16)] = plsc.load_gather(idx_v, [d])
            sd, sl = plsc.sort_key_val(d, pos)
            sh_v[pl.ds(16, 16)] = sd
            prev = sh_v[pl.ds(15, 16)]
            nxt = sh_v[pl.ds(17, 16)]
            head = sd != prev
            tail = sd != nxt
            hp = plsc.cummax(jnp.where(head, pos, 0))
            r = pos - hp
            old = plsc.load_gather(next_v, [sd])
            plsc.store_scatter(next_v, [sd], old + r + 1, mask=tail)
            ch = i // 8
            kk = i % 8
            plsc.store_scatter(
                slot_v, [jnp.full((16,), ch, jnp.int32), kk * 16 + sl],
                old + r)
            return c

        lax.fori_loop(0, _NV, vstep, 0)

        def pump(ids_v, out_hbm):
            def fireg(ci, c):
                off = ci * 128
                pltpu.async_copy(tbl_hbm.at[ids_v.at[pl.ds(off, 128)]],
                                 rows_v.at[pl.ds(off, 128), :], sem)
                return c

            lax.fori_loop(0, _NCH, fireg, 0)
            pltpu.make_async_copy(
                tbl_hbm.at[pl.ds(0, _EPW), :], rows_v, sem).wait()

            def fires(ci, c):
                off = ci * 128
                pltpu.async_copy(rows_v.at[pl.ds(off, 128), :],
                                 out_hbm.at[slot_v.at[ci]], sem)
                return c

            lax.fori_loop(0, _NCH, fires, 0)
            pltpu.make_async_copy(
                out_hbm.at[pl.ds(0, _EPW), :], rows_v, sem).wait()

        pump(s_v, a_hbm)
        pump(c_v, b_hbm)

    return k(dstp, srcp, tbase, tbl16, idxp)


def _group_meta(poff8f, e0f, e1f, e2f):
    def body(po_ref, e0_ref, e1_ref, e2_ref, gs_ref, vc_ref):
        i = pl.program_id(0)
        po = po_ref[...]
        g8 = ((lax.broadcasted_iota(jnp.int32, (256, 1), 0)
               + i * 256) * 8).astype(jnp.float32)
        cmp = jnp.where(po <= g8, 1.0, 0.0)
        gseg = jnp.sum(cmp, axis=1, keepdims=True) - 1.0
        segid = lax.broadcasted_iota(jnp.int32, (1, SPAD), 1).astype(jnp.float32)
        oh = jnp.where(segid == gseg, 1.0, 0.0)
        d0 = jnp.dot(oh, e0_ref[...], preferred_element_type=jnp.float32)
        d1 = jnp.dot(oh, e1_ref[...], preferred_element_type=jnp.float32)
        d2 = jnp.dot(oh, e2_ref[...], preferred_element_type=jnp.float32)
        endsel = (d2 * 128.0 + d1) * 128.0 + d0
        gs_ref[...] = gseg.astype(jnp.int32)[None]
        vc_ref[...] = jnp.clip(endsel - g8, 0.0, 8.0)[None]

    ngb = NG // 256
    return pl.pallas_call(
        body,
        grid=(ngb,),
        in_specs=[
            pl.BlockSpec((1, SPAD), lambda i: (0, 0)),
            pl.BlockSpec((SPAD, 1), lambda i: (0, 0)),
            pl.BlockSpec((SPAD, 1), lambda i: (0, 0)),
            pl.BlockSpec((SPAD, 1), lambda i: (0, 0)),
        ],
        out_specs=(
            pl.BlockSpec((1, 256, 1), lambda i: (i, 0, 0)),
            pl.BlockSpec((1, 256, 1), lambda i: (i, 0, 0)),
        ),
        out_shape=(
            jax.ShapeDtypeStruct((ngb, 256, 1), jnp.int32),
            jax.ShapeDtypeStruct((ngb, 256, 1), jnp.float32),
        ),
    )(poff8f, e0f, e1f, e2f)


BROW = 1024
BGRP = BROW // 8


def _mlp_segmax(a, b, vrow, gkeys, w1, b1, w2, b2, w3, b3):
    def body(keys_ref, a_ref, b_ref, v_ref, w1_ref, b1_ref, w2_ref, b2_ref,
             w3_ref, b3_ref, out_ref, gm_ref):
        i = pl.program_id(0)

        @pl.when(i == 0)
        def _init():
            out_ref[...] = jnp.full((SPAD, 512), -jnp.inf, jnp.float32)

        msg = a_ref[...] - b_ref[...]
        h = jnp.dot(msg, w1_ref[...], preferred_element_type=jnp.float32)
        h = h + b1_ref[...]
        h = jnp.where(h > 0, h, 0.2 * h)
        h = jnp.dot(h, w2_ref[...], preferred_element_type=jnp.float32)
        h = h + b2_ref[...]
        h = jnp.where(h > 0, h, 0.2 * h)
        z = jnp.dot(h, w3_ref[...], preferred_element_type=jnp.float32)
        z = z + b3_ref[...]
        z = jnp.where(z > 0, z, 0.2 * z)
        z3 = z.reshape(BGRP, 8, 512)
        l3 = lax.broadcasted_iota(jnp.int32, (BGRP, 8, 1), 1).astype(jnp.float32)
        vc3 = v_ref[...].reshape(BGRP, 1, 1)
        z3 = jnp.where(l3 < vc3, z3, -jnp.inf)
        gm_ref[...] = jnp.max(z3, axis=1)

        def upd(j, carry):
            d = keys_ref[0, 0, j]
            row = gm_ref[pl.ds(j, 1), :]
            out_ref[pl.ds(d, 1), :] = jnp.maximum(out_ref[pl.ds(d, 1), :], row)
            return carry

        lax.fori_loop(0, BGRP, upd, 0)

    return pl.pallas_call(
        body,
        grid=(EPAD // BROW,),
        in_specs=[
            pl.BlockSpec((1, 1, BGRP), lambda i: (i, 0, 0),
                         memory_space=pltpu.SMEM),
            pl.BlockSpec((BROW, 16), lambda i: (i, 0)),
            pl.BlockSpec((BROW, 16), lambda i: (i, 0)),
            pl.BlockSpec((BGRP, 1), lambda i: (i, 0)),
            pl.BlockSpec((16, 64), lambda i: (0, 0)),
            pl.BlockSpec((1, 64), lambda i: (0, 0)),
            pl.BlockSpec((64, 128), lambda i: (0, 0)),
            pl.BlockSpec((1, 128), lambda i: (0, 0)),
            pl.BlockSpec((128, 512), lambda i: (0, 0)),
            pl.BlockSpec((1, 512), lambda i: (0, 0)),
        ],
        out_specs=pl.BlockSpec((SPAD, 512), lambda i: (0, 0)),
        out_shape=jax.ShapeDtypeStruct((SPAD, 512), jnp.float32),
        scratch_shapes=[pltpu.VMEM((BGRP, 512), jnp.float32)],
        compiler_params=pltpu.CompilerParams(
            dimension_semantics=("arbitrary",)),
    )(gkeys, a, b, vrow, w1, b1, w2, b2, w3, b3)


def _encode_select(aggraw, poss, ew1a, ew1b, eb1, ew2, eb2):
    def body(agg_ref, poss_ref, w1a_ref, w1b_ref, b1_ref, w2_ref, b2_ref,
             z_ref, mi_ref, is_ref, sb_ref):
        raw = agg_ref[...]
        agg = jnp.where(raw < -1e30, 0.0, raw)
        p16 = poss_ref[...]
        h2 = (jnp.dot(agg, w1a_ref[...], preferred_element_type=jnp.float32)
              + jnp.dot(p16, w1b_ref[...], preferred_element_type=jnp.float32)
              + b1_ref[...])
        h2 = jnp.where(h2 > 0, h2, 0.2 * h2)
        out = jnp.dot(h2, w2_ref[...], preferred_element_type=jnp.float32)
        out = out + b2_ref[...]
        mean = out[:, :512]
        lv = out[:, 512:]
        invstd = jnp.exp(-0.5 * lv)
        std = jnp.exp(0.5 * lv)
        mi_ref[...] = mean * invstd
        is_ref[...] = invstd
        score = jnp.mean(std, axis=1, keepdims=True)
        rowid = lax.broadcasted_iota(jnp.int32, (SPAD, 1), 0)
        bcol = p16[:, 3:4]
        valid = rowid < S
        for bb in range(B):
            act0 = jnp.logical_and(valid, bcol == float(bb))
            sb_ref[...] = jnp.where(act0, score, jnp.inf)

            def rnd(r, carry):
                nu, de = carry
                cur = sb_ref[...]
                m = jnp.min(cur)
                cand = jnp.where(cur == m, rowid, jnp.int32(2 ** 30))
                isel = jnp.min(cand)
                nu = nu + mi_ref[pl.ds(isel, 1), :]
                de = de + is_ref[pl.ds(isel, 1), :]
                sb_ref[pl.ds(isel, 1), :] = jnp.full((1, 1), jnp.inf,
                                                     jnp.float32)
                return nu, de

            nu, de = lax.fori_loop(
                0, NC, rnd,
                (jnp.zeros((1, 512), jnp.float32),
                 jnp.zeros((1, 512), jnp.float32)))
            z_ref[bb:bb + 1, :] = nu / de

    return pl.pallas_call(
        body,
        grid=(1,),
        in_specs=[
            pl.BlockSpec((SPAD, 512), lambda i: (0, 0)),
            pl.BlockSpec((SPAD, 16), lambda i: (0, 0)),
            pl.BlockSpec((512, 512), lambda i: (0, 0)),
            pl.BlockSpec((16, 512), lambda i: (0, 0)),
            pl.BlockSpec((1, 512), lambda i: (0, 0)),
            pl.BlockSpec((512, 1024), lambda i: (0, 0)),
            pl.BlockSpec((1, 1024), lambda i: (0, 0)),
        ],
        out_specs=pl.BlockSpec((8, 512), lambda i: (0, 0)),
        out_shape=jax.ShapeDtypeStruct((8, 512), jnp.float32),
        scratch_shapes=[
            pltpu.VMEM((SPAD, 512), jnp.float32),
            pltpu.VMEM((SPAD, 512), jnp.float32),
            pltpu.VMEM((SPAD, 1), jnp.float32),
        ],
        compiler_params=pltpu.CompilerParams(
            vmem_limit_bytes=100 * 1024 * 1024),
    )(aggraw, poss, ew1a, ew1b, eb1, ew2, eb2)


def _decode(z, gg, f1w1a, f1w1g, f1b1, f1w2, f1b2, f1w3p, f1b3p,
            f2w1a, f2w1b, f2b1, f2w2, f2b2, f2w3p, f2b3p):
    def body(z_ref, gg_ref, w1a, w1g, bb1, w12, b12, w13, b13,
             w2a, w2b, bb2, w22, b22, w23, b23, out_ref):
        zf = z_ref[...]
        gg_ = gg_ref[...]
        bcol = gg_[:, 3:4]
        code = jnp.zeros((256, 512), jnp.float32)
        for bb in range(B):
            code = jnp.where(bcol == float(bb), zf[bb:bb + 1, :], code)
        h = (jnp.dot(code, w1a[...], preferred_element_type=jnp.float32)
             + jnp.dot(gg_, w1g[...], preferred_element_type=jnp.float32)
             + bb1[...])
        h = jnp.maximum(h, 0.0)
        h = jnp.dot(h, w12[...], preferred_element_type=jnp.float32) + b12[...]
        h = jnp.maximum(h, 0.0)
        x1 = jnp.dot(h, w13[...], preferred_element_type=jnp.float32) + b13[...]
        h = (jnp.dot(code, w2a[...], preferred_element_type=jnp.float32)
             + jnp.dot(x1, w2b[...], preferred_element_type=jnp.float32)
             + bb2[...])
        h = jnp.maximum(h, 0.0)
        h = jnp.dot(h, w22[...], preferred_element_type=jnp.float32) + b22[...]
        h = jnp.maximum(h, 0.0)
        out_ref[...] = (jnp.dot(h, w23[...], preferred_element_type=jnp.float32)
                        + b23[...])

    wspec = lambda r, c: pl.BlockSpec((r, c), lambda i: (0, 0))
    return pl.pallas_call(
        body,
        grid=(DBLK,),
        in_specs=[
            pl.BlockSpec((8, 512), lambda i: (0, 0)),
            pl.BlockSpec((256, 16), lambda i: (i, 0)),
            wspec(512, 512), wspec(16, 512), wspec(1, 512),
            wspec(512, 512), wspec(1, 512),
            wspec(512, 16), wspec(1, 16),
            wspec(512, 512), wspec(16, 512), wspec(1, 512),
            wspec(512, 512), wspec(1, 512),
            wspec(512, 16), wspec(1, 16),
        ],
        out_specs=pl.BlockSpec((256, 16), lambda i: (i, 0)),
        out_shape=jax.ShapeDtypeStruct((DPAD, 16), jnp.float32),
        compiler_params=pltpu.CompilerParams(
            dimension_semantics=("arbitrary",)),
    )(z, gg, f1w1a, f1w1g, f1b1, f1w2, f1b2, f1w3p, f1b3p,
      f2w1a, f2w1b, f2b1, f2w2, f2b2, f2w3p, f2b3p)


def _np_grid16():
    g1, g2 = np.meshgrid(np.linspace(-0.3, 0.3, 45), np.linspace(-0.3, 0.3, 45))
    g = np.stack([g1.reshape(-1), g2.reshape(-1)], axis=-1).astype(np.float32)
    gg = np.zeros((DPAD, 16), np.float32)
    t = np.arange(DPAD)
    bcol = np.minimum(t // G, B - 1)
    gg[:B * G, 0:2] = np.tile(g, (B, 1))
    gg[:, 3] = bcol
    return gg


_GG16_NP = _np_grid16()


def kernel(pos, batch, idx, src, dst, lW1, lb1, lW2, lb2, lW3, lb3,
           eW1, eb1, eW2, eb2,
           f1W1, f1b1, f1W2, f1b2, f1W3, f1b3,
           f2W1, f2b1, f2W2, f2b2, f2W3, f2b3):
    i32 = jnp.int32
    dstp = jnp.concatenate([dst.astype(i32),
                            jnp.full((EP - E,), SPAD - 1, i32)])
    srcp = jnp.concatenate([src.astype(i32), jnp.zeros((EP - E,), i32)])
    idxp = jnp.concatenate([idx.astype(i32), jnp.zeros((SPAD - S,), i32)])
    tbl16 = jnp.concatenate(
        [pos, batch.astype(jnp.float32)[:, None],
         jnp.zeros((N, 12), jnp.float32)], axis=1)

    hists, poss = _sc_hist(dstp, tbl16, idxp)

    counts = jnp.sum(hists, axis=0)
    pc8 = (counts + 7) // 8 * 8
    csum = jnp.cumsum(pc8)
    poff8 = jnp.concatenate([jnp.zeros((1,), i32), csum[:-1]])
    tbase = poff8[None, :] + (jnp.cumsum(hists, axis=0) - hists)
    poff8f = poff8.astype(jnp.float32).reshape(1, SPAD)
    endv = poff8 + counts
    e0f = (endv % 128).astype(jnp.float32).reshape(SPAD, 1)
    e1f = (endv // 128 % 128).astype(jnp.float32).reshape(SPAD, 1)
    e2f = (endv // 16384).astype(jnp.float32).reshape(SPAD, 1)
    gsegb, vcntb = _group_meta(poff8f, e0f, e1f, e2f)
    vcol = vcntb.reshape(NG, 1)
    gkeys = gsegb.reshape(EPAD // BROW, 1, BGRP)

    a, b = _sc_permute(dstp, srcp, tbase, tbl16, idxp)

    w1 = jnp.zeros((16, 64), jnp.float32).at[:3].set(lW1)
    aggraw = _mlp_segmax(a, b, vcol, gkeys, w1, lb1[None, :], lW2,
                         lb2[None, :], lW3, lb3[None, :])

    ew1a = eW1[:512]
    ew1b = jnp.zeros((16, 512), jnp.float32).at[:3].set(eW1[512:515])
    z = _encode_select(aggraw, poss, ew1a, ew1b, eb1[None, :], eW2,
                       eb2[None, :])

    f1w1a = f1W1[:512]
    f1w1g = jnp.zeros((16, 512), jnp.float32).at[:2].set(f1W1[512:514])
    f1w3p = jnp.zeros((512, 16), jnp.float32).at[:, :3].set(f1W3)
    f1b3p = jnp.zeros((1, 16), jnp.float32).at[0, :3].set(f1b3)
    f2w1a = f2W1[:512]
    f2w1b = jnp.zeros((16, 512), jnp.float32).at[:3].set(f2W1[512:515])
    f2w3p = jnp.zeros((512, 16), jnp.float32).at[:, :3].set(f2W3)
    f2b3p = jnp.zeros((1, 16), jnp.float32).at[0, :3].set(f2b3)
    outp = _decode(z, jnp.asarray(_GG16_NP), f1w1a, f1w1g, f1b1[None, :],
                   f1W2, f1b2[None, :], f1w3p, f1b3p, f2w1a, f2w1b,
                   f2b1[None, :], f2W2, f2b2[None, :], f2w3p, f2b3p)
    return outp[:B * G, :3].reshape(B, G, 3)


# speedup vs baseline: 29.4088x; 1.0923x over previous
import functools

import jax
import jax.numpy as jnp
import numpy as np
from jax import lax
from jax.experimental import pallas as pl
from jax.experimental.pallas import tpu as pltpu
from jax.experimental.pallas import tpu_sc as plsc

N = 10000
B = 8
S = 2000
E = 160000
K = 512
NC = 10
G = 45 * 45

SPAD = 2048
EP = 163840
EPAD = 178176
NBLK = EPAD // 256
NG = EPAD // 8
DPAD = 16384
DBLK = DPAD // 256

_NWORK = 32
_EPW = EP // _NWORK
_NV = _EPW // 16
_NCH = _EPW // 128
_SPW = SPAD // _NWORK

_MESH = dict(core_axis_name="c", subcore_axis_name="s")


def _sc_hist(dstp, tbl16, idxp):
    mesh = plsc.VectorSubcoreMesh(**_MESH)

    @functools.partial(
        pl.kernel,
        out_type=(
            jax.ShapeDtypeStruct((_NWORK, SPAD), jnp.int32),
            jax.ShapeDtypeStruct((SPAD, 16), jnp.float32),
        ),
        mesh=mesh,
        scratch_types=[
            pltpu.VMEM((_EPW,), jnp.int32),
            pltpu.VMEM((16 * SPAD,), jnp.int32),
            pltpu.VMEM((SPAD,), jnp.int32),
            pltpu.VMEM((_SPW, 16), jnp.float32),
            pltpu.VMEM((_SPW,), jnp.int32),
            pltpu.SemaphoreType.DMA,
        ],
        compiler_params=pltpu.CompilerParams(use_tc_tiling_on_sc=False, needs_layout_passes=False),
    )
    def k(dst_hbm, tbl_hbm, idxp_hbm, hist_hbm, poss_hbm,
          d_v, bank_v, tot_v, ps_v, pi_v, sem):
        wid = lax.axis_index("s") * 2 + lax.axis_index("c")
        base = wid * _EPW
        pltpu.sync_copy(dst_hbm.at[pl.ds(base, _EPW)], d_v)
        zero16 = jnp.zeros((16,), jnp.int32)

        def mz(i, c):
            bank_v[pl.ds(i * 16, 16)] = zero16
            return c

        lax.fori_loop(0, SPAD, mz, 0)
        lanes = lax.iota(jnp.int32, 16) * SPAD

        def acc(i, c):
            d = d_v[pl.ds(i * 16, 16)]
            addr = lanes + d
            old = plsc.load_gather(bank_v, [addr])
            plsc.store_scatter(bank_v, [addr], old + 1)
            return c

        lax.fori_loop(0, _NV, acc, 0)

        def red(j, c):
            sacc = jnp.zeros((16,), jnp.int32)
            for l in range(16):
                sacc = sacc + bank_v[pl.ds(l * SPAD + j * 16, 16)]
            tot_v[pl.ds(j * 16, 16)] = sacc
            return c

        lax.fori_loop(0, SPAD // 16, red, 0)
        pltpu.sync_copy(tot_v, hist_hbm.at[wid])

        sbase = wid * _SPW
        pltpu.sync_copy(idxp_hbm.at[pl.ds(sbase, _SPW)], pi_v)
        pltpu.async_copy(tbl_hbm.at[pi_v], ps_v, sem).wait()
        pltpu.sync_copy(ps_v, poss_hbm.at[pl.ds(sbase, _SPW), :])

    return k(dstp, tbl16, idxp)


def _sc_permute(dstp, srcp, tbase, tbl16, idxp):
    mesh = plsc.VectorSubcoreMesh(**_MESH)

    @functools.partial(
        pl.kernel,
        out_type=(
            jax.ShapeDtypeStruct((EPAD, 16), jnp.float32),
            jax.ShapeDtypeStruct((EPAD, 16), jnp.float32),
        ),
        mesh=mesh,
        scratch_types=[
            pltpu.VMEM((_EPW,), jnp.int32),
            pltpu.VMEM((_EPW,), jnp.int32),
            pltpu.VMEM((_EPW,), jnp.int32),
            pltpu.VMEM((_NCH, 128), jnp.int32),
            pltpu.VMEM((SPAD,), jnp.int32),
            pltpu.VMEM((SPAD,), jnp.int32),
            pltpu.VMEM((48,), jnp.int32),
            pltpu.VMEM((_EPW, 16), jnp.float32),
            pltpu.SemaphoreType.DMA,
        ],
        compiler_params=pltpu.CompilerParams(use_tc_tiling_on_sc=False, needs_layout_passes=False),
    )
    def k(dst_hbm, src_hbm, tb_hbm, tbl_hbm, idxp_hbm, a_hbm, b_hbm,
          d_v, s_v, c_v, slot_v, next_v, idx_v, sh_v, rows_v, sem):
        wid = lax.axis_index("s") * 2 + lax.axis_index("c")
        base = wid * _EPW
        pltpu.sync_copy(dst_hbm.at[pl.ds(base, _EPW)], d_v)
        pltpu.sync_copy(src_hbm.at[pl.ds(base, _EPW)], s_v)
        pltpu.sync_copy(tb_hbm.at[wid], next_v)
        pltpu.sync_copy(idxp_hbm, idx_v)
        neg16 = jnp.full((16,), -1, jnp.int32)
        sh_v[pl.ds(0, 16)] = neg16
        sh_v[pl.ds(32, 16)] = neg16
        pos = lax.iota(jnp.int32, 16)

        def vstep(i, c):
            d = d_v[pl.ds(i * 16, 16)]
            c_v[pl.ds(i * 16, 16)] = plsc.load_gather(idx_v, [d])
            sd, sl = plsc.sort_key_val(d, pos)
            sh_v[pl.ds(16, 16)] = sd
            prev = sh_v[pl.ds(15, 16)]
            nxt = sh_v[pl.ds(17, 16)]
            head = sd != prev
            tail = sd != nxt
            hp = plsc.cummax(jnp.where(head, pos, 0))
            r = pos - hp
            old = plsc.load_gather(next_v, [sd])
            plsc.store_scatter(next_v, [sd], old + r + 1, mask=tail)
            ch = i // 8
            kk = i % 8
            plsc.store_scatter(
                slot_v, [jnp.full((16,), ch, jnp.int32), kk * 16 + sl],
                old + r)
            return c

        lax.fori_loop(0, _NV, vstep, 0)

        def pump(ids_v, out_hbm):
            def fireg(ci, c):
                off = ci * 128
                pltpu.async_copy(tbl_hbm.at[ids_v.at[pl.ds(off, 128)]],
                                 rows_v.at[pl.ds(off, 128), :], sem)
                return c

            lax.fori_loop(0, _NCH, fireg, 0)
            pltpu.make_async_copy(
                tbl_hbm.at[pl.ds(0, _EPW), :], rows_v, sem).wait()

            def fires(ci, c):
                off = ci * 128
                pltpu.async_copy(rows_v.at[pl.ds(off, 128), :],
                                 out_hbm.at[slot_v.at[ci]], sem)
                return c

            lax.fori_loop(0, _NCH, fires, 0)
            pltpu.make_async_copy(
                out_hbm.at[pl.ds(0, _EPW), :], rows_v, sem).wait()

        pump(s_v, a_hbm)
        pump(c_v, b_hbm)

    return k(dstp, srcp, tbase, tbl16, idxp)


def _group_meta(poff8f, e0f, e1f, e2f):
    def body(po_ref, e0_ref, e1_ref, e2_ref, gs_ref, vc_ref):
        i = pl.program_id(0)
        po = po_ref[...]
        g8 = ((lax.broadcasted_iota(jnp.int32, (256, 1), 0)
               + i * 256) * 8).astype(jnp.float32)
        cmp = jnp.where(po <= g8, 1.0, 0.0)
        gseg = jnp.sum(cmp, axis=1, keepdims=True) - 1.0
        segid = lax.broadcasted_iota(jnp.int32, (1, SPAD), 1).astype(jnp.float32)
        oh = jnp.where(segid == gseg, 1.0, 0.0)
        d0 = jnp.dot(oh, e0_ref[...], preferred_element_type=jnp.float32)
        d1 = jnp.dot(oh, e1_ref[...], preferred_element_type=jnp.float32)
        d2 = jnp.dot(oh, e2_ref[...], preferred_element_type=jnp.float32)
        endsel = (d2 * 128.0 + d1) * 128.0 + d0
        gs_ref[...] = gseg.astype(jnp.int32)[None]
        vc_ref[...] = jnp.clip(endsel - g8, 0.0, 8.0)[None]

    ngb = NG // 256
    return pl.pallas_call(
        body,
        grid=(ngb,),
        in_specs=[
            pl.BlockSpec((1, SPAD), lambda i: (0, 0)),
            pl.BlockSpec((SPAD, 1), lambda i: (0, 0)),
            pl.BlockSpec((SPAD, 1), lambda i: (0, 0)),
            pl.BlockSpec((SPAD, 1), lambda i: (0, 0)),
        ],
        out_specs=(
            pl.BlockSpec((1, 256, 1), lambda i: (i, 0, 0)),
            pl.BlockSpec((1, 256, 1), lambda i: (i, 0, 0)),
        ),
        out_shape=(
            jax.ShapeDtypeStruct((ngb, 256, 1), jnp.int32),
            jax.ShapeDtypeStruct((ngb, 256, 1), jnp.float32),
        ),
    )(poff8f, e0f, e1f, e2f)


BROW = 1024
BGRP = BROW // 8


def _mlp_segmax(a, b, vrow, gkeys, w1, b1, w2, b2, w3, b3):
    def body(keys_ref, a_ref, b_ref, v_ref, w1_ref, b1_ref, w2_ref, b2_ref,
             w3_ref, b3_ref, out_ref, gm_ref, ac0, ac1, ac2, ac3):
        i = pl.program_id(0)
        accs = (ac0, ac1, ac2, ac3)

        @pl.when(i == 0)
        def _init():
            for acc in accs:
                acc[...] = jnp.full((SPAD, 512), -jnp.inf, jnp.float32)

        msg = a_ref[...] - b_ref[...]
        h = jnp.dot(msg, w1_ref[...], preferred_element_type=jnp.float32)
        h = h + b1_ref[...]
        h = jnp.where(h > 0, h, 0.2 * h)
        h = jnp.dot(h, w2_ref[...], preferred_element_type=jnp.float32)
        h = h + b2_ref[...]
        h = jnp.where(h > 0, h, 0.2 * h)
        z = jnp.dot(h, w3_ref[...], preferred_element_type=jnp.float32)
        z = z + b3_ref[...]
        z = jnp.where(z > 0, z, 0.2 * z)
        z3 = z.reshape(BGRP, 8, 512)
        l3 = lax.broadcasted_iota(jnp.int32, (BGRP, 8, 1), 1).astype(jnp.float32)
        vc3 = v_ref[...].reshape(BGRP, 1, 1)
        z3 = jnp.where(l3 < vc3, z3, -jnp.inf)
        gm_ref[...] = jnp.max(z3, axis=1)

        q = BGRP // 4

        def upd(j, carry):
            for qq in range(4):
                d = keys_ref[j + qq * q, 0]
                row = gm_ref[pl.ds(j + qq * q, 1), :]
                acc = accs[qq]
                acc[pl.ds(d, 1), :] = jnp.maximum(acc[pl.ds(d, 1), :], row)
            return carry

        lax.fori_loop(0, q, upd, 0)

        @pl.when(i == EPAD // BROW - 1)
        def _fin():
            out_ref[...] = jnp.maximum(
                jnp.maximum(accs[0][...], accs[1][...]),
                jnp.maximum(accs[2][...], accs[3][...]))

    return pl.pallas_call(
        body,
        grid=(EPAD // BROW,),
        in_specs=[
            pl.BlockSpec((BGRP, 1), lambda i: (i, 0),
                         memory_space=pltpu.SMEM),
            pl.BlockSpec((BROW, 16), lambda i: (i, 0)),
            pl.BlockSpec((BROW, 16), lambda i: (i, 0)),
            pl.BlockSpec((BGRP, 1), lambda i: (i, 0)),
            pl.BlockSpec((16, 64), lambda i: (0, 0)),
            pl.BlockSpec((1, 64), lambda i: (0, 0)),
            pl.BlockSpec((64, 128), lambda i: (0, 0)),
            pl.BlockSpec((1, 128), lambda i: (0, 0)),
            pl.BlockSpec((128, 512), lambda i: (0, 0)),
            pl.BlockSpec((1, 512), lambda i: (0, 0)),
        ],
        out_specs=pl.BlockSpec((SPAD, 512), lambda i: (0, 0)),
        out_shape=jax.ShapeDtypeStruct((SPAD, 512), jnp.float32),
        scratch_shapes=[pltpu.VMEM((BGRP, 512), jnp.float32),
                        pltpu.VMEM((SPAD, 512), jnp.float32),
                        pltpu.VMEM((SPAD, 512), jnp.float32),
                        pltpu.VMEM((SPAD, 512), jnp.float32),
                        pltpu.VMEM((SPAD, 512), jnp.float32)],
        compiler_params=pltpu.CompilerParams(
            dimension_semantics=("arbitrary",),
            vmem_limit_bytes=100 * 1024 * 1024),
    )(gkeys, a, b, vrow, w1, b1, w2, b2, w3, b3)


def _encode_select(aggraw, poss, ew1a, ew1b, eb1, ew2, eb2):
    def body(agg_ref, poss_ref, w1a_ref, w1b_ref, b1_ref, w2_ref, b2_ref,
             z_ref, mi_ref, is_ref, sb_ref):
        raw = agg_ref[...]
        agg = jnp.where(raw < -1e30, 0.0, raw)
        p16 = poss_ref[...]
        h2 = (jnp.dot(agg, w1a_ref[...], preferred_element_type=jnp.float32)
              + jnp.dot(p16, w1b_ref[...], preferred_element_type=jnp.float32)
              + b1_ref[...])
        h2 = jnp.where(h2 > 0, h2, 0.2 * h2)
        out = jnp.dot(h2, w2_ref[...], preferred_element_type=jnp.float32)
        out = out + b2_ref[...]
        mean = out[:, :512]
        lv = out[:, 512:]
        invstd = jnp.exp(-0.5 * lv)
        std = jnp.exp(0.5 * lv)
        mi_ref[...] = mean * invstd
        is_ref[...] = invstd
        score = jnp.mean(std, axis=1, keepdims=True)
        rowid = lax.broadcasted_iota(jnp.int32, (SPAD, 1), 0)
        bcol = p16[:, 3:4]
        valid = rowid < S
        for bb in range(B):
            act0 = jnp.logical_and(valid, bcol == float(bb))
            sb_ref[...] = jnp.where(act0, score, jnp.inf)

            def rnd(r, carry):
                nu, de = carry
                cur = sb_ref[...]
                m = jnp.min(cur)
                cand = jnp.where(cur == m, rowid, jnp.int32(2 ** 30))
                isel = jnp.min(cand)
                nu = nu + mi_ref[pl.ds(isel, 1), :]
                de = de + is_ref[pl.ds(isel, 1), :]
                sb_ref[pl.ds(isel, 1), :] = jnp.full((1, 1), jnp.inf,
                                                     jnp.float32)
                return nu, de

            nu, de = lax.fori_loop(
                0, NC, rnd,
                (jnp.zeros((1, 512), jnp.float32),
                 jnp.zeros((1, 512), jnp.float32)))
            z_ref[bb:bb + 1, :] = nu / de

    return pl.pallas_call(
        body,
        grid=(1,),
        in_specs=[
            pl.BlockSpec((SPAD, 512), lambda i: (0, 0)),
            pl.BlockSpec((SPAD, 16), lambda i: (0, 0)),
            pl.BlockSpec((512, 512), lambda i: (0, 0)),
            pl.BlockSpec((16, 512), lambda i: (0, 0)),
            pl.BlockSpec((1, 512), lambda i: (0, 0)),
            pl.BlockSpec((512, 1024), lambda i: (0, 0)),
            pl.BlockSpec((1, 1024), lambda i: (0, 0)),
        ],
        out_specs=pl.BlockSpec((8, 512), lambda i: (0, 0)),
        out_shape=jax.ShapeDtypeStruct((8, 512), jnp.float32),
        scratch_shapes=[
            pltpu.VMEM((SPAD, 512), jnp.float32),
            pltpu.VMEM((SPAD, 512), jnp.float32),
            pltpu.VMEM((SPAD, 1), jnp.float32),
        ],
        compiler_params=pltpu.CompilerParams(
            vmem_limit_bytes=100 * 1024 * 1024),
    )(aggraw, poss, ew1a, ew1b, eb1, ew2, eb2)


def _decode(z, gg, f1w1a, f1w1g, f1b1, f1w2, f1b2, f1w3p, f1b3p,
            f2w1a, f2w1b, f2b1, f2w2, f2b2, f2w3p, f2b3p):
    def body(z_ref, gg_ref, w1a, w1g, bb1, w12, b12, w13, b13,
             w2a, w2b, bb2, w22, b22, w23, b23, out_ref):
        zf = z_ref[...]
        gg_ = gg_ref[...]
        bcol = gg_[:, 3:4]
        code = jnp.zeros((256, 512), jnp.float32)
        for bb in range(B):
            code = jnp.where(bcol == float(bb), zf[bb:bb + 1, :], code)
        h = (jnp.dot(code, w1a[...], preferred_element_type=jnp.float32)
             + jnp.dot(gg_, w1g[...], preferred_element_type=jnp.float32)
             + bb1[...])
        h = jnp.maximum(h, 0.0)
        h = jnp.dot(h, w12[...], preferred_element_type=jnp.float32) + b12[...]
        h = jnp.maximum(h, 0.0)
        x1 = jnp.dot(h, w13[...], preferred_element_type=jnp.float32) + b13[...]
        h = (jnp.dot(code, w2a[...], preferred_element_type=jnp.float32)
             + jnp.dot(x1, w2b[...], preferred_element_type=jnp.float32)
             + bb2[...])
        h = jnp.maximum(h, 0.0)
        h = jnp.dot(h, w22[...], preferred_element_type=jnp.float32) + b22[...]
        h = jnp.maximum(h, 0.0)
        out_ref[...] = (jnp.dot(h, w23[...], preferred_element_type=jnp.float32)
                        + b23[...])

    wspec = lambda r, c: pl.BlockSpec((r, c), lambda i: (0, 0))
    return pl.pallas_call(
        body,
        grid=(DBLK,),
        in_specs=[
            pl.BlockSpec((8, 512), lambda i: (0, 0)),
            pl.BlockSpec((256, 16), lambda i: (i, 0)),
            wspec(512, 512), wspec(16, 512), wspec(1, 512),
            wspec(512, 512), wspec(1, 512),
            wspec(512, 16), wspec(1, 16),
            wspec(512, 512), wspec(16, 512), wspec(1, 512),
            wspec(512, 512), wspec(1, 512),
            wspec(512, 16), wspec(1, 16),
        ],
        out_specs=pl.BlockSpec((256, 16), lambda i: (i, 0)),
        out_shape=jax.ShapeDtypeStruct((DPAD, 16), jnp.float32),
        compiler_params=pltpu.CompilerParams(
            dimension_semantics=("arbitrary",)),
    )(z, gg, f1w1a, f1w1g, f1b1, f1w2, f1b2, f1w3p, f1b3p,
      f2w1a, f2w1b, f2b1, f2w2, f2b2, f2w3p, f2b3p)


def _np_grid16():
    g1, g2 = np.meshgrid(np.linspace(-0.3, 0.3, 45), np.linspace(-0.3, 0.3, 45))
    g = np.stack([g1.reshape(-1), g2.reshape(-1)], axis=-1).astype(np.float32)
    gg = np.zeros((DPAD, 16), np.float32)
    t = np.arange(DPAD)
    bcol = np.minimum(t // G, B - 1)
    gg[:B * G, 0:2] = np.tile(g, (B, 1))
    gg[:, 3] = bcol
    return gg


_GG16_NP = _np_grid16()


def kernel(pos, batch, idx, src, dst, lW1, lb1, lW2, lb2, lW3, lb3,
           eW1, eb1, eW2, eb2,
           f1W1, f1b1, f1W2, f1b2, f1W3, f1b3,
           f2W1, f2b1, f2W2, f2b2, f2W3, f2b3):
    i32 = jnp.int32
    dstp = jnp.concatenate([dst.astype(i32),
                            jnp.full((EP - E,), SPAD - 1, i32)])
    srcp = jnp.concatenate([src.astype(i32), jnp.zeros((EP - E,), i32)])
    idxp = jnp.concatenate([idx.astype(i32), jnp.zeros((SPAD - S,), i32)])
    tbl16 = jnp.concatenate(
        [pos, batch.astype(jnp.float32)[:, None],
         jnp.zeros((N, 12), jnp.float32)], axis=1)

    hists, poss = _sc_hist(dstp, tbl16, idxp)

    counts = jnp.sum(hists, axis=0)
    pc8 = (counts + 7) // 8 * 8
    csum = jnp.cumsum(pc8)
    poff8 = jnp.concatenate([jnp.zeros((1,), i32), csum[:-1]])
    tbase = poff8[None, :] + (jnp.cumsum(hists, axis=0) - hists)
    poff8f = poff8.astype(jnp.float32).reshape(1, SPAD)
    endv = poff8 + counts
    e0f = (endv % 128).astype(jnp.float32).reshape(SPAD, 1)
    e1f = (endv // 128 % 128).astype(jnp.float32).reshape(SPAD, 1)
    e2f = (endv // 16384).astype(jnp.float32).reshape(SPAD, 1)
    gsegb, vcntb = _group_meta(poff8f, e0f, e1f, e2f)
    vcol = vcntb.reshape(NG, 1)
    gkeys = gsegb.reshape(NG, 1)

    a, b = _sc_permute(dstp, srcp, tbase, tbl16, idxp)

    w1 = jnp.zeros((16, 64), jnp.float32).at[:3].set(lW1)
    aggraw = _mlp_segmax(a, b, vcol, gkeys, w1, lb1[None, :], lW2,
                         lb2[None, :], lW3, lb3[None, :])

    ew1a = eW1[:512]
    ew1b = jnp.zeros((16, 512), jnp.float32).at[:3].set(eW1[512:515])
    z = _encode_select(aggraw, poss, ew1a, ew1b, eb1[None, :], eW2,
                       eb2[None, :])

    f1w1a = f1W1[:512]
    f1w1g = jnp.zeros((16, 512), jnp.float32).at[:2].set(f1W1[512:514])
    f1w3p = jnp.zeros((512, 16), jnp.float32).at[:, :3].set(f1W3)
    f1b3p = jnp.zeros((1, 16), jnp.float32).at[0, :3].set(f1b3)
    f2w1a = f2W1[:512]
    f2w1b = jnp.zeros((16, 512), jnp.float32).at[:3].set(f2W1[512:515])
    f2w3p = jnp.zeros((512, 16), jnp.float32).at[:, :3].set(f2W3)
    f2b3p = jnp.zeros((1, 16), jnp.float32).at[0, :3].set(f2b3)
    outp = _decode(z, jnp.asarray(_GG16_NP), f1w1a, f1w1g, f1b1[None, :],
                   f1W2, f1b2[None, :], f1w3p, f1b3p, f2w1a, f2w1b,
                   f2b1[None, :], f2W2, f2b2[None, :], f2w3p, f2b3p)
    return outp[:B * G, :3].reshape(B, G, 3)
